# Initial kernel scaffold; baseline (speedup 1.0000x reference)
import jax
import jax.numpy as jnp
from jax.experimental import pallas as pl


def kernel(x, edge_index, batch, W_l, W_r, b1, gamma, beta, W_lin, b_lin):
    raise NotImplementedError("write your pallas kernel here")


# speedup vs baseline: 60.7004x; 60.7004x over previous
import functools

import jax
import jax.numpy as jnp
from jax import lax
from jax.experimental import pallas as pl
from jax.experimental.pallas import tpu as pltpu
from jax.experimental.pallas import tpu_sc as plsc

N = 10000
E = 320000
D = 128
H = 128
OUT = 2
G = 64
K = 10
ALPHA = 0.1

NC = 2
NS = 16
NW = NC * NS

EW1 = E // NW
C1 = 80
NCH1 = EW1 // C1
NSL1 = N // NS

ET2 = E // NS
EG2 = ET2 // 16
N2 = 10240
NSL2 = N2 // NS
NR2 = N2 // 128
TR2 = NR2 // NS


def _rc(idx):
    return lax.shift_right_logical(idx, 7), lax.bitwise_and(idx, 127)


def _sage_kernel(x_hbm, src_hbm, dst_hbm, zeros_hbm,
                 aggp_out, degp_out,
                 srcb, dstb, rows, degv, aggsh, sem):
    c = lax.axis_index("c")
    s = lax.axis_index("s")
    wid = s * NC + c

    pltpu.sync_copy(zeros_hbm.at[pl.ds(s * NSL1, NSL1)],
                    aggsh.at[pl.ds(s * NSL1, NSL1)])

    def zero_deg(j, _):
        degv[j // 8, pl.ds((j % 8) * 16, 16)] = jnp.zeros((16,), jnp.float32)
        return 0
    lax.fori_loop(0, N2 // 16, zero_deg, 0)

    pltpu.sync_copy(src_hbm.at[wid], srcb)
    pltpu.sync_copy(dst_hbm.at[wid], dstb)

    plsc.subcore_barrier()

    ones16 = jnp.ones((16,), jnp.float32)

    def chunk(j, _):
        pltpu.async_copy(x_hbm.at[srcb.at[j]], rows, sem).wait()
        pltpu.sync_copy(rows, aggsh.at[dstb.at[j]], add=True)
        for g in range(C1 // 16):
            idx = dstb[j, pl.ds(g * 16, 16)]
            dr, dc = _rc(idx)
            plsc.addupdate_scatter(degv, [dr, dc], ones16)
        return 0

    lax.fori_loop(0, NCH1, chunk, 0)

    plsc.subcore_barrier()

    pltpu.sync_copy(aggsh.at[pl.ds(s * NSL1, NSL1)],
                    aggp_out.at[c, pl.ds(s * NSL1, NSL1)])
    pltpu.sync_copy(degv, degp_out.at[wid])


def _sage_sc(x, src3, dst3, zeros_big):
    mesh = plsc.VectorSubcoreMesh(core_axis_name="c", subcore_axis_name="s",
                                  num_cores=NC, num_subcores=NS)
    f = pl.kernel(
        _sage_kernel,
        out_type=(jax.ShapeDtypeStruct((NC, N, D), jnp.float32),
                  jax.ShapeDtypeStruct((NW, NR2, 128), jnp.float32)),
        mesh=mesh,
        compiler_params=pltpu.CompilerParams(use_tc_tiling_on_sc=False, needs_layout_passes=False),
        scratch_types=[
            pltpu.VMEM((NCH1, C1), jnp.int32),
            pltpu.VMEM((NCH1, C1), jnp.int32),
            pltpu.VMEM((C1, D), jnp.float32),
            pltpu.VMEM((NR2, 128), jnp.float32),
            pltpu.VMEM_SHARED((N, D), jnp.float32),
            pltpu.SemaphoreType.DMA,
        ],
    )
    return f(x, src3, dst3, zeros_big)


def _dense_a_body(aggp_ref, x_ref, degt_ref, wl_ref, wr_ref, b1_ref,
                  hpre_ref, stats_ref):
    i = pl.program_id(0)
    deg = jnp.sum(degt_ref[...], axis=1)
    agg = (aggp_ref[0] + aggp_ref[1]) / jnp.maximum(deg, 1.0)[:, None]
    h = lax.dot_general(agg, wl_ref[...], (((1,), (1,)), ((), ())),
                        preferred_element_type=jnp.float32)
    h = h + lax.dot_general(x_ref[...], wr_ref[...], (((1,), (1,)), ((), ())),
                            preferred_element_type=jnp.float32)
    h = h + b1_ref[0:1, :]
    hpre_ref[...] = h
    bs = jnp.sum(h, axis=0, keepdims=True)
    bq = jnp.sum(h * h, axis=0, keepdims=True)
    upd = jnp.concatenate([bs, bq, jnp.zeros((6, 128), jnp.float32)], axis=0)

    @pl.when(i == 0)
    def _():
        stats_ref[...] = upd

    @pl.when(i > 0)
    def _():
        stats_ref[...] = stats_ref[...] + upd


def _dense_a(aggp, x, degt, W_l, W_r, b1_b):
    R = 1000
    grid = (N // R,)
    return pl.pallas_call(
        _dense_a_body,
        grid=grid,
        in_specs=[
            pl.BlockSpec((NC, R, D), lambda i: (0, i, 0)),
            pl.BlockSpec((R, D), lambda i: (i, 0)),
            pl.BlockSpec((R, NW), lambda i: (i, 0)),
            pl.BlockSpec((H, D), lambda i: (0, 0)),
            pl.BlockSpec((H, D), lambda i: (0, 0)),
            pl.BlockSpec((8, H), lambda i: (0, 0)),
        ],
        out_specs=[
            pl.BlockSpec((R, H), lambda i: (i, 0)),
            pl.BlockSpec((8, H), lambda i: (0, 0)),
        ],
        out_shape=[
            jax.ShapeDtypeStruct((N, H), jnp.float32),
            jax.ShapeDtypeStruct((8, H), jnp.float32),
        ],
    )(aggp, x, degt, W_l, W_r, b1_b)


def _dense_b_body(hpre_ref, stats_ref, degt_ref, gamma_ref, beta_ref,
                  wlin_ref, out_ref):
    stats = stats_ref[...]
    mean = stats[0:1, :] / N
    var = stats[1:2, :] / N - mean * mean
    inv = lax.rsqrt(var + 1e-5)
    hn = (hpre_ref[...] - mean) * (inv * gamma_ref[0:1, :]) + beta_ref[0:1, :]
    hn = jnp.maximum(hn, 0.0)
    r8 = lax.dot_general(wlin_ref[...], hn, (((1,), (1,)), ((), ())),
                         preferred_element_type=jnp.float32)
    deg = jnp.sum(degt_ref[...], axis=1)
    d2 = 1.0 / (deg + 1.0)
    d1 = jnp.sqrt(d2)
    row = lax.broadcasted_iota(jnp.int32, r8.shape, 0)
    outb = jnp.where(row == 2, d1[None, :],
                     jnp.where(row == 3, d2[None, :], r8))
    out_ref[...] = outb.reshape(out_ref.shape)


def _dense_b(hpre, stats, degt, gamma_b, beta_b, wlin_pad):
    R = 1000
    grid = (N // R,)
    return pl.pallas_call(
        _dense_b_body,
        grid=grid,
        in_specs=[
            pl.BlockSpec((R, H), lambda i: (i, 0)),
            pl.BlockSpec((8, H), lambda i: (0, 0)),
            pl.BlockSpec((R, NW), lambda i: (i, 0)),
            pl.BlockSpec((8, H), lambda i: (0, 0)),
            pl.BlockSpec((8, H), lambda i: (0, 0)),
            pl.BlockSpec((8, H), lambda i: (0, 0)),
        ],
        out_specs=pl.BlockSpec((1, 8, R), lambda i: (i, 0, 0)),
        out_shape=jax.ShapeDtypeStruct((N // R, 8, R), jnp.float32),
    )(hpre, stats, degt, gamma_b, beta_b, wlin_pad)


def _plane_slice(ref, j):
    return ref[j // 8, pl.ds((j % 8) * 16, 16)]


def _appnp_kernel(src_hbm, dst_hbm, planes_hbm, batch_hbm,
                  pool_out, cnt_out,
                  srcv, dstv, normv, zv, rv, accv, wv, tv, batchv,
                  rowidx, binp, binc, zaccsh, znewsh):
    c = lax.axis_index("c")
    s = lax.axis_index("s")

    pltpu.sync_copy(src_hbm.at[s], srcv)
    pltpu.sync_copy(dst_hbm.at[s], dstv)
    pltpu.sync_copy(planes_hbm.at[2], wv)
    pltpu.sync_copy(planes_hbm.at[c], rv)
    pltpu.sync_copy(planes_hbm.at[c], zv)
    pltpu.sync_copy(batch_hbm, batchv)

    for g in range(NR2 // 16):
        rowidx[pl.ds(g * 16, 16)] = lax.iota(jnp.int32, 16) + g * 16

    def norm_body(j, _):
        sl = pl.ds(j * 16, 16)
        sr, sc_ = _rc(srcv[sl])
        dr, dc = _rc(dstv[sl])
        a = plsc.load_gather(wv, [sr, sc_])
        b = plsc.load_gather(wv, [dr, dc])
        normv[sl] = a * b
        return 0
    lax.fori_loop(0, EG2, norm_body, 0)

    pltpu.sync_copy(planes_hbm.at[3], wv)

    zeros16 = jnp.zeros((16,), jnp.float32)

    def zero_acc(j, _):
        accv[j // 8, pl.ds((j % 8) * 16, 16)] = zeros16
        return 0
    lax.fori_loop(0, N2 // 16, zero_acc, 0)
    rb0 = s * TR2
    pltpu.sync_copy(accv.at[pl.ds(0, TR2)], zaccsh.at[pl.ds(rb0, TR2)])
    plsc.subcore_barrier()

    omal = jnp.float32(1.0 - ALPHA)
    al = jnp.float32(ALPHA)

    for _ in range(K):
        def edge_body(j, _):
            sl = pl.ds(j * 16, 16)
            sr, sc_ = _rc(srcv[sl])
            dr, dc = _rc(dstv[sl])
            vals = plsc.load_gather(zv, [sr, sc_]) * normv[sl]
            plsc.addupdate_scatter(accv, [dr, dc], vals)
            return 0
        lax.fori_loop(0, EG2, edge_body, 0)

        pltpu.sync_copy(accv, zaccsh.at[rowidx], add=True)
        plsc.subcore_barrier()

        pltpu.sync_copy(zaccsh.at[pl.ds(rb0, TR2)], tv)

        lax.fori_loop(0, N2 // 16, zero_acc, 0)
        pltpu.sync_copy(accv.at[pl.ds(0, TR2)], zaccsh.at[pl.ds(rb0, TR2)])

        def upd_body(j, _):
            jg = s * (NSL2 // 16) + j
            znew = (omal * (_plane_slice(tv, j)
                            + _plane_slice(wv, jg) * _plane_slice(zv, jg))
                    + al * _plane_slice(rv, jg))
            tv[j // 8, pl.ds((j % 8) * 16, 16)] = znew
            return 0
        lax.fori_loop(0, NSL2 // 16, upd_body, 0)

        pltpu.sync_copy(tv, znewsh.at[pl.ds(rb0, TR2)])
        plsc.subcore_barrier()
        pltpu.sync_copy(znewsh, zv)

    @pl.when(s == 0)
    def _():
        for g in range(G // 16):
            binp[g, :] = jnp.zeros((16,), jnp.float32)
            binc[g, :] = jnp.zeros((16,), jnp.float32)

        ones16 = jnp.ones((16,), jnp.float32)

        def pool_body(j, _):
            sl = pl.ds(j * 16, 16)
            bi = batchv[sl]
            br = lax.shift_right_logical(bi, 4)
            bc = lax.bitwise_and(bi, 15)
            plsc.addupdate_scatter(binp, [br, bc], _plane_slice(zv, j))
            plsc.addupdate_scatter(binc, [br, bc], ones16)
            return 0
        lax.fori_loop(0, N // 16, pool_body, 0)

        pltpu.sync_copy(binp, pool_out.at[c])

        @pl.when(c == 0)
        def _():
            pltpu.sync_copy(binc, cnt_out)


def _appnp_sc(src2, dst2, planes, batch):
    mesh = plsc.VectorSubcoreMesh(core_axis_name="c", subcore_axis_name="s",
                                  num_cores=NC, num_subcores=NS)
    f = pl.kernel(
        _appnp_kernel,
        out_type=(jax.ShapeDtypeStruct((NC, G // 16, 16), jnp.float32),
                  jax.ShapeDtypeStruct((G // 16, 16), jnp.float32)),
        mesh=mesh,
        compiler_params=pltpu.CompilerParams(use_tc_tiling_on_sc=False, needs_layout_passes=False),
        scratch_types=[
            pltpu.VMEM((ET2,), jnp.int32),
            pltpu.VMEM((ET2,), jnp.int32),
            pltpu.VMEM((ET2,), jnp.float32),
            pltpu.VMEM((NR2, 128), jnp.float32),
            pltpu.VMEM((NR2, 128), jnp.float32),
            pltpu.VMEM((NR2, 128), jnp.float32),
            pltpu.VMEM((NR2, 128), jnp.float32),
            pltpu.VMEM((TR2, 128), jnp.float32),
            pltpu.VMEM((N,), jnp.int32),
            pltpu.VMEM((NR2,), jnp.int32),
            pltpu.VMEM((G // 16, 16), jnp.float32),
            pltpu.VMEM((G // 16, 16), jnp.float32),
            pltpu.VMEM_SHARED((NR2, 128), jnp.float32),
            pltpu.VMEM_SHARED((NR2, 128), jnp.float32),
        ],
    )
    return f(src2, dst2, planes, batch)


def kernel(x, edge_index, batch, W_l, W_r, b1, gamma, beta, W_lin, b_lin):
    src = edge_index[0]
    dst = edge_index[1]

    src3 = src.reshape(NW, NCH1, C1)
    dst3 = dst.reshape(NW, NCH1, C1)
    zeros_big = jnp.zeros((N, D), jnp.float32)

    aggp, degp = _sage_sc(x, src3, dst3, zeros_big)
    degt = degp.reshape(NW, N2)[:, :N].T

    b1_b = jnp.broadcast_to(b1[None, :], (8, H))
    hpre, stats = _dense_a(aggp, x, degt, W_l, W_r, b1_b)

    gamma_b = jnp.broadcast_to(gamma[None, :], (8, H))
    beta_b = jnp.broadcast_to(beta[None, :], (8, H))
    wlin_pad = jnp.zeros((8, H), jnp.float32).at[:OUT].set(W_lin)
    planes3 = _dense_b(hpre, stats, degt, gamma_b, beta_b, wlin_pad)

    planes = planes3.transpose(1, 0, 2).reshape(8, N)
    planes = jnp.pad(planes, ((0, 0), (0, N2 - N))).reshape(8, NR2, 128)

    src2 = src.reshape(NS, ET2)
    dst2 = dst.reshape(NS, ET2)
    pool, cnt = _appnp_sc(src2, dst2, planes, batch)
    pool = pool.reshape(NC, G)
    cnt = cnt.reshape(G)

    out = (pool / jnp.clip(cnt, 1.0, None)[None, :]).T + b_lin[None, :]
    return out


# speedup vs baseline: 79.0660x; 1.3026x over previous
import functools

import jax
import jax.numpy as jnp
from jax import lax
from jax.experimental import pallas as pl
from jax.experimental.pallas import tpu as pltpu
from jax.experimental.pallas import tpu_sc as plsc

N = 10000
E = 320000
D = 128
H = 128
OUT = 2
G = 64
K = 10
ALPHA = 0.1

NC = 2
NS = 16
NW = NC * NS

EW1 = E // NW
C1 = 80
NCH1 = EW1 // C1
NSL1 = N // NS

ET2 = E // NS
EG2 = ET2 // 16
N2 = 10240
NSL2 = N2 // NS
NR2 = N2 // 128
TR2 = NR2 // NS


def _rc(idx):
    return lax.shift_right_logical(idx, 7), lax.bitwise_and(idx, 127)


def _sage_kernel(x_hbm, src_hbm, dst_hbm, zeros_hbm,
                 aggp_out, degp_out,
                 srcb, dstb, rows, degv, aggsh, sems):
    c = lax.axis_index("c")
    s = lax.axis_index("s")
    wid = s * NC + c

    pltpu.sync_copy(zeros_hbm.at[pl.ds(s * NSL1, NSL1)],
                    aggsh.at[pl.ds(s * NSL1, NSL1)])

    def zero_deg(j, _):
        degv[j // 8, pl.ds((j % 8) * 16, 16)] = jnp.zeros((16,), jnp.float32)
        return 0
    lax.fori_loop(0, N2 // 16, zero_deg, 0)

    pltpu.sync_copy(src_hbm.at[wid], srcb)
    pltpu.sync_copy(dst_hbm.at[wid], dstb)

    plsc.subcore_barrier()

    ones16 = jnp.ones((16,), jnp.float32)
    NB = 2

    for b in range(NB):
        pltpu.async_copy(x_hbm.at[srcb.at[b]], rows.at[b], sems.at[b])

    def chunk_group(i, _):
        for b in range(NB):
            j = i * NB + b
            pltpu.make_async_copy(x_hbm.at[srcb.at[j]], rows.at[b],
                                  sems.at[b]).wait()
            pltpu.sync_copy(rows.at[b], aggsh.at[dstb.at[j]], add=True)
            jn = j + NB
            @pl.when(jn < NCH1)
            def _():
                pltpu.async_copy(x_hbm.at[srcb.at[jn]], rows.at[b], sems.at[b])
            for g in range(C1 // 16):
                idx = dstb[j, pl.ds(g * 16, 16)]
                dr, dc = _rc(idx)
                plsc.addupdate_scatter(degv, [dr, dc], ones16)
        return 0

    lax.fori_loop(0, NCH1 // NB, chunk_group, 0)

    for j in range((NCH1 // NB) * NB, NCH1):
        b = j % NB
        pltpu.make_async_copy(x_hbm.at[srcb.at[j]], rows.at[b], sems.at[b]).wait()
        pltpu.sync_copy(rows.at[b], aggsh.at[dstb.at[j]], add=True)
        for g in range(C1 // 16):
            idx = dstb[j, pl.ds(g * 16, 16)]
            dr, dc = _rc(idx)
            plsc.addupdate_scatter(degv, [dr, dc], ones16)

    plsc.subcore_barrier()

    pltpu.sync_copy(aggsh.at[pl.ds(s * NSL1, NSL1)],
                    aggp_out.at[c, pl.ds(s * NSL1, NSL1)])
    pltpu.sync_copy(degv, degp_out.at[wid])


def _sage_sc(x, src3, dst3, zeros_big):
    mesh = plsc.VectorSubcoreMesh(core_axis_name="c", subcore_axis_name="s",
                                  num_cores=NC, num_subcores=NS)
    f = pl.kernel(
        _sage_kernel,
        out_type=(jax.ShapeDtypeStruct((NC, N, D), jnp.float32),
                  jax.ShapeDtypeStruct((NW, NR2, 128), jnp.float32)),
        mesh=mesh,
        compiler_params=pltpu.CompilerParams(use_tc_tiling_on_sc=False, needs_layout_passes=False),
        scratch_types=[
            pltpu.VMEM((NCH1, C1), jnp.int32),
            pltpu.VMEM((NCH1, C1), jnp.int32),
            pltpu.VMEM((2, C1, D), jnp.float32),
            pltpu.VMEM((NR2, 128), jnp.float32),
            pltpu.VMEM_SHARED((N, D), jnp.float32),
            pltpu.SemaphoreType.DMA((2,)),
        ],
    )
    return f(x, src3, dst3, zeros_big)


def _dense_a_body(aggp_ref, x_ref, degt_ref, wl_ref, wr_ref, b1_ref,
                  hpre_ref, stats_ref):
    i = pl.program_id(0)
    deg = jnp.sum(degt_ref[...], axis=1)
    agg = (aggp_ref[0] + aggp_ref[1]) / jnp.maximum(deg, 1.0)[:, None]
    h = lax.dot_general(agg, wl_ref[...], (((1,), (1,)), ((), ())),
                        preferred_element_type=jnp.float32)
    h = h + lax.dot_general(x_ref[...], wr_ref[...], (((1,), (1,)), ((), ())),
                            preferred_element_type=jnp.float32)
    h = h + b1_ref[0:1, :]
    hpre_ref[...] = h
    bs = jnp.sum(h, axis=0, keepdims=True)
    bq = jnp.sum(h * h, axis=0, keepdims=True)
    upd = jnp.concatenate([bs, bq, jnp.zeros((6, 128), jnp.float32)], axis=0)

    @pl.when(i == 0)
    def _():
        stats_ref[...] = upd

    @pl.when(i > 0)
    def _():
        stats_ref[...] = stats_ref[...] + upd


def _dense_a(aggp, x, degt, W_l, W_r, b1_b):
    R = 1000
    grid = (N // R,)
    return pl.pallas_call(
        _dense_a_body,
        grid=grid,
        in_specs=[
            pl.BlockSpec((NC, R, D), lambda i: (0, i, 0)),
            pl.BlockSpec((R, D), lambda i: (i, 0)),
            pl.BlockSpec((R, NW), lambda i: (i, 0)),
            pl.BlockSpec((H, D), lambda i: (0, 0)),
            pl.BlockSpec((H, D), lambda i: (0, 0)),
            pl.BlockSpec((8, H), lambda i: (0, 0)),
        ],
        out_specs=[
            pl.BlockSpec((R, H), lambda i: (i, 0)),
            pl.BlockSpec((8, H), lambda i: (0, 0)),
        ],
        out_shape=[
            jax.ShapeDtypeStruct((N, H), jnp.float32),
            jax.ShapeDtypeStruct((8, H), jnp.float32),
        ],
    )(aggp, x, degt, W_l, W_r, b1_b)


def _dense_b_body(hpre_ref, stats_ref, degt_ref, gamma_ref, beta_ref,
                  wlin_ref, out_ref):
    stats = stats_ref[...]
    mean = stats[0:1, :] / N
    var = stats[1:2, :] / N - mean * mean
    inv = lax.rsqrt(var + 1e-5)
    hn = (hpre_ref[...] - mean) * (inv * gamma_ref[0:1, :]) + beta_ref[0:1, :]
    hn = jnp.maximum(hn, 0.0)
    r8 = lax.dot_general(wlin_ref[...], hn, (((1,), (1,)), ((), ())),
                         preferred_element_type=jnp.float32)
    deg = jnp.sum(degt_ref[...], axis=1)
    d2 = 1.0 / (deg + 1.0)
    d1 = jnp.sqrt(d2)
    row = lax.broadcasted_iota(jnp.int32, r8.shape, 0)
    outb = jnp.where(row == 2, d1[None, :],
                     jnp.where(row == 3, d2[None, :], r8))
    out_ref[...] = outb.reshape(out_ref.shape)


def _dense_b(hpre, stats, degt, gamma_b, beta_b, wlin_pad):
    R = 1000
    grid = (N // R,)
    return pl.pallas_call(
        _dense_b_body,
        grid=grid,
        in_specs=[
            pl.BlockSpec((R, H), lambda i: (i, 0)),
            pl.BlockSpec((8, H), lambda i: (0, 0)),
            pl.BlockSpec((R, NW), lambda i: (i, 0)),
            pl.BlockSpec((8, H), lambda i: (0, 0)),
            pl.BlockSpec((8, H), lambda i: (0, 0)),
            pl.BlockSpec((8, H), lambda i: (0, 0)),
        ],
        out_specs=pl.BlockSpec((1, 8, R), lambda i: (i, 0, 0)),
        out_shape=jax.ShapeDtypeStruct((N // R, 8, R), jnp.float32),
    )(hpre, stats, degt, gamma_b, beta_b, wlin_pad)


def _plane_slice(ref, j):
    return ref[j // 8, pl.ds((j % 8) * 16, 16)]


def _appnp_kernel(src_hbm, dst_hbm, planes_hbm, batch_hbm,
                  pool_out, cnt_out,
                  srcv, dstv, normv, zv, rv, accv, wv, tv, batchv,
                  rowidx, binp, binc, zaccsh, znewsh):
    c = lax.axis_index("c")
    s = lax.axis_index("s")

    pltpu.sync_copy(src_hbm.at[s], srcv)
    pltpu.sync_copy(dst_hbm.at[s], dstv)
    pltpu.sync_copy(planes_hbm.at[2], wv)
    pltpu.sync_copy(planes_hbm.at[c], rv)
    pltpu.sync_copy(planes_hbm.at[c], zv)
    pltpu.sync_copy(batch_hbm, batchv)

    for g in range(NR2 // 16):
        rowidx[pl.ds(g * 16, 16)] = lax.iota(jnp.int32, 16) + g * 16

    def norm_body(j, _):
        for u in range(5):
            sl = pl.ds(j * 80 + u * 16, 16)
            sr, sc_ = _rc(srcv[sl])
            dr, dc = _rc(dstv[sl])
            a = plsc.load_gather(wv, [sr, sc_])
            b = plsc.load_gather(wv, [dr, dc])
            normv[sl] = a * b
        return 0
    lax.fori_loop(0, EG2 // 5, norm_body, 0)

    pltpu.sync_copy(planes_hbm.at[3], wv)

    zeros16 = jnp.zeros((16,), jnp.float32)

    def zero_acc(j, _):
        for g in range(8):
            accv[j, pl.ds(g * 16, 16)] = zeros16
        return 0
    lax.fori_loop(0, NR2, zero_acc, 0)
    rb0 = s * TR2
    pltpu.sync_copy(accv.at[pl.ds(0, TR2)], zaccsh.at[pl.ds(rb0, TR2)])
    plsc.subcore_barrier()

    omal = jnp.float32(1.0 - ALPHA)
    al = jnp.float32(ALPHA)

    for _ in range(K):
        def edge_body(j, _):
            for u in range(5):
                sl = pl.ds(j * 80 + u * 16, 16)
                sr, sc_ = _rc(srcv[sl])
                dr, dc = _rc(dstv[sl])
                vals = plsc.load_gather(zv, [sr, sc_]) * normv[sl]
                plsc.addupdate_scatter(accv, [dr, dc], vals)
            return 0
        lax.fori_loop(0, EG2 // 5, edge_body, 0)

        pltpu.sync_copy(accv, zaccsh.at[rowidx], add=True)
        plsc.subcore_barrier()

        pltpu.sync_copy(zaccsh.at[pl.ds(rb0, TR2)], tv)

        lax.fori_loop(0, NR2, zero_acc, 0)
        pltpu.sync_copy(accv.at[pl.ds(0, TR2)], zaccsh.at[pl.ds(rb0, TR2)])

        def upd_body(j, _):
            jg = rb0 + j
            for g in range(8):
                cs = pl.ds(g * 16, 16)
                znew = (omal * (tv[j, cs] + wv[jg, cs] * zv[jg, cs])
                        + al * rv[jg, cs])
                tv[j, cs] = znew
            return 0
        lax.fori_loop(0, TR2, upd_body, 0)

        pltpu.sync_copy(tv, znewsh.at[pl.ds(rb0, TR2)])
        plsc.subcore_barrier()
        pltpu.sync_copy(znewsh, zv)

    @pl.when(s == 0)
    def _():
        for g in range(G // 16):
            binp[g, :] = jnp.zeros((16,), jnp.float32)
            binc[g, :] = jnp.zeros((16,), jnp.float32)

        ones16 = jnp.ones((16,), jnp.float32)

        def pool_body(j, _):
            for u in range(5):
                jf = j * 5 + u
                sl = pl.ds(jf * 16, 16)
                bi = batchv[sl]
                br = lax.shift_right_logical(bi, 4)
                bc = lax.bitwise_and(bi, 15)
                plsc.addupdate_scatter(binp, [br, bc], _plane_slice(zv, jf))
                plsc.addupdate_scatter(binc, [br, bc], ones16)
            return 0
        lax.fori_loop(0, N // 16 // 5, pool_body, 0)

        pltpu.sync_copy(binp, pool_out.at[c])

        @pl.when(c == 0)
        def _():
            pltpu.sync_copy(binc, cnt_out)


def _appnp_sc(src2, dst2, planes, batch):
    mesh = plsc.VectorSubcoreMesh(core_axis_name="c", subcore_axis_name="s",
                                  num_cores=NC, num_subcores=NS)
    f = pl.kernel(
        _appnp_kernel,
        out_type=(jax.ShapeDtypeStruct((NC, G // 16, 16), jnp.float32),
                  jax.ShapeDtypeStruct((G // 16, 16), jnp.float32)),
        mesh=mesh,
        compiler_params=pltpu.CompilerParams(use_tc_tiling_on_sc=False, needs_layout_passes=False),
        scratch_types=[
            pltpu.VMEM((ET2,), jnp.int32),
            pltpu.VMEM((ET2,), jnp.int32),
            pltpu.VMEM((ET2,), jnp.float32),
            pltpu.VMEM((NR2, 128), jnp.float32),
            pltpu.VMEM((NR2, 128), jnp.float32),
            pltpu.VMEM((NR2, 128), jnp.float32),
            pltpu.VMEM((NR2, 128), jnp.float32),
            pltpu.VMEM((TR2, 128), jnp.float32),
            pltpu.VMEM((N,), jnp.int32),
            pltpu.VMEM((NR2,), jnp.int32),
            pltpu.VMEM((G // 16, 16), jnp.float32),
            pltpu.VMEM((G // 16, 16), jnp.float32),
            pltpu.VMEM_SHARED((NR2, 128), jnp.float32),
            pltpu.VMEM_SHARED((NR2, 128), jnp.float32),
        ],
    )
    return f(src2, dst2, planes, batch)


def kernel(x, edge_index, batch, W_l, W_r, b1, gamma, beta, W_lin, b_lin):
    src = edge_index[0]
    dst = edge_index[1]

    src3 = src.reshape(NW, NCH1, C1)
    dst3 = dst.reshape(NW, NCH1, C1)
    zeros_big = jnp.zeros((N, D), jnp.float32)

    aggp, degp = _sage_sc(x, src3, dst3, zeros_big)
    degt = degp.reshape(NW, N2)[:, :N].T

    b1_b = jnp.broadcast_to(b1[None, :], (8, H))
    hpre, stats = _dense_a(aggp, x, degt, W_l, W_r, b1_b)

    gamma_b = jnp.broadcast_to(gamma[None, :], (8, H))
    beta_b = jnp.broadcast_to(beta[None, :], (8, H))
    wlin_pad = jnp.zeros((8, H), jnp.float32).at[:OUT].set(W_lin)
    planes3 = _dense_b(hpre, stats, degt, gamma_b, beta_b, wlin_pad)

    planes = planes3.transpose(1, 0, 2).reshape(8, N)
    planes = jnp.pad(planes, ((0, 0), (0, N2 - N))).reshape(8, NR2, 128)

    src2 = src.reshape(NS, ET2)
    dst2 = dst.reshape(NS, ET2)
    pool, cnt = _appnp_sc(src2, dst2, planes, batch)
    pool = pool.reshape(NC, G)
    cnt = cnt.reshape(G)

    out = (pool / jnp.clip(cnt, 1.0, None)[None, :]).T + b_lin[None, :]
    return out


# speedup vs baseline: 104.4377x; 1.3209x over previous
import functools

import jax
import jax.numpy as jnp
from jax import lax
from jax.experimental import pallas as pl
from jax.experimental.pallas import tpu as pltpu
from jax.experimental.pallas import tpu_sc as plsc

N = 10000
E = 320000
D = 128
H = 128
OUT = 2
G = 64
K = 10
ALPHA = 0.1

NC = 2
NS = 16
NW = NC * NS

EW1 = E // NW
C1 = 80
NCH1 = EW1 // C1
NSL1 = N // NS

ET2 = E // NS
EG2 = ET2 // 16
N2 = 10240
NSL2 = N2 // NS
NR2 = N2 // 128
TR2 = NR2 // NS


def _rc(idx):
    return lax.shift_right_logical(idx, 7), lax.bitwise_and(idx, 127)


def _sage_kernel(x_hbm, src_hbm, dst_hbm, zeros_hbm,
                 aggp_out, degp_out,
                 srcb, dstb, rows, degv, aggsh, sems):
    c = lax.axis_index("c")
    s = lax.axis_index("s")
    wid = s * NC + c

    pltpu.sync_copy(zeros_hbm.at[pl.ds(s * NSL1, NSL1)],
                    aggsh.at[pl.ds(s * NSL1, NSL1)])

    def zero_deg(j, _):
        degv[j // 8, pl.ds((j % 8) * 16, 16)] = jnp.zeros((16,), jnp.float32)
        return 0
    lax.fori_loop(0, N2 // 16, zero_deg, 0)

    pltpu.sync_copy(src_hbm.at[wid], srcb)
    pltpu.sync_copy(dst_hbm.at[wid], dstb)

    plsc.subcore_barrier()

    ones16 = jnp.ones((16,), jnp.float32)
    NB = 2

    for b in range(NB):
        pltpu.async_copy(x_hbm.at[srcb.at[b]], rows.at[b], sems.at[b])

    def chunk_group(i, _):
        for b in range(NB):
            j = i * NB + b
            pltpu.make_async_copy(x_hbm.at[srcb.at[j]], rows.at[b],
                                  sems.at[b]).wait()
            pltpu.sync_copy(rows.at[b], aggsh.at[dstb.at[j]], add=True)
            jn = j + NB
            @pl.when(jn < NCH1)
            def _():
                pltpu.async_copy(x_hbm.at[srcb.at[jn]], rows.at[b], sems.at[b])
            for g in range(C1 // 16):
                idx = dstb[j, pl.ds(g * 16, 16)]
                dr, dc = _rc(idx)
                plsc.addupdate_scatter(degv, [dr, dc], ones16)
        return 0

    lax.fori_loop(0, NCH1 // NB, chunk_group, 0)

    for j in range((NCH1 // NB) * NB, NCH1):
        b = j % NB
        pltpu.make_async_copy(x_hbm.at[srcb.at[j]], rows.at[b], sems.at[b]).wait()
        pltpu.sync_copy(rows.at[b], aggsh.at[dstb.at[j]], add=True)
        for g in range(C1 // 16):
            idx = dstb[j, pl.ds(g * 16, 16)]
            dr, dc = _rc(idx)
            plsc.addupdate_scatter(degv, [dr, dc], ones16)

    plsc.subcore_barrier()

    pltpu.sync_copy(aggsh.at[pl.ds(s * NSL1, NSL1)],
                    aggp_out.at[c, pl.ds(s * NSL1, NSL1)])
    pltpu.sync_copy(degv, degp_out.at[wid])


def _sage_sc(x, src3, dst3, zeros_big):
    mesh = plsc.VectorSubcoreMesh(core_axis_name="c", subcore_axis_name="s",
                                  num_cores=NC, num_subcores=NS)
    f = pl.kernel(
        _sage_kernel,
        out_type=(jax.ShapeDtypeStruct((NC, N, D), jnp.float32),
                  jax.ShapeDtypeStruct((NW, NR2, 128), jnp.float32)),
        mesh=mesh,
        compiler_params=pltpu.CompilerParams(use_tc_tiling_on_sc=False, needs_layout_passes=False),
        scratch_types=[
            pltpu.VMEM((NCH1, C1), jnp.int32),
            pltpu.VMEM((NCH1, C1), jnp.int32),
            pltpu.VMEM((2, C1, D), jnp.float32),
            pltpu.VMEM((NR2, 128), jnp.float32),
            pltpu.VMEM_SHARED((N, D), jnp.float32),
            pltpu.SemaphoreType.DMA((2,)),
        ],
    )
    return f(x, src3, dst3, zeros_big)


def _dense_a_body(aggp_ref, x_ref, degt_ref, wl_ref, wr_ref, b1_ref,
                  hpre_ref, stats_ref):
    i = pl.program_id(0)
    deg = jnp.sum(degt_ref[...], axis=1)
    agg = (aggp_ref[0] + aggp_ref[1]) / jnp.maximum(deg, 1.0)[:, None]
    h = lax.dot_general(agg, wl_ref[...], (((1,), (1,)), ((), ())),
                        preferred_element_type=jnp.float32)
    h = h + lax.dot_general(x_ref[...], wr_ref[...], (((1,), (1,)), ((), ())),
                            preferred_element_type=jnp.float32)
    h = h + b1_ref[0:1, :]
    hpre_ref[...] = h
    bs = jnp.sum(h, axis=0, keepdims=True)
    bq = jnp.sum(h * h, axis=0, keepdims=True)
    upd = jnp.concatenate([bs, bq, jnp.zeros((6, 128), jnp.float32)], axis=0)

    @pl.when(i == 0)
    def _():
        stats_ref[...] = upd

    @pl.when(i > 0)
    def _():
        stats_ref[...] = stats_ref[...] + upd


def _dense_a(aggp, x, degt, W_l, W_r, b1_b):
    R = 1000
    grid = (N // R,)
    return pl.pallas_call(
        _dense_a_body,
        grid=grid,
        in_specs=[
            pl.BlockSpec((NC, R, D), lambda i: (0, i, 0)),
            pl.BlockSpec((R, D), lambda i: (i, 0)),
            pl.BlockSpec((R, NW), lambda i: (i, 0)),
            pl.BlockSpec((H, D), lambda i: (0, 0)),
            pl.BlockSpec((H, D), lambda i: (0, 0)),
            pl.BlockSpec((8, H), lambda i: (0, 0)),
        ],
        out_specs=[
            pl.BlockSpec((R, H), lambda i: (i, 0)),
            pl.BlockSpec((8, H), lambda i: (0, 0)),
        ],
        out_shape=[
            jax.ShapeDtypeStruct((N, H), jnp.float32),
            jax.ShapeDtypeStruct((8, H), jnp.float32),
        ],
    )(aggp, x, degt, W_l, W_r, b1_b)


def _dense_b_body(hpre_ref, stats_ref, degt_ref, gamma_ref, beta_ref,
                  wlin_ref, out_ref):
    stats = stats_ref[...]
    mean = stats[0:1, :] / N
    var = stats[1:2, :] / N - mean * mean
    inv = lax.rsqrt(var + 1e-5)
    hn = (hpre_ref[...] - mean) * (inv * gamma_ref[0:1, :]) + beta_ref[0:1, :]
    hn = jnp.maximum(hn, 0.0)
    r8 = lax.dot_general(wlin_ref[...], hn, (((1,), (1,)), ((), ())),
                         preferred_element_type=jnp.float32)
    deg = jnp.sum(degt_ref[...], axis=1)
    d2 = 1.0 / (deg + 1.0)
    d1 = jnp.sqrt(d2)
    row = lax.broadcasted_iota(jnp.int32, r8.shape, 0)
    outb = jnp.where(row == 2, d1[None, :],
                     jnp.where(row == 3, d2[None, :], r8))
    out_ref[...] = outb.reshape(out_ref.shape)


def _dense_b(hpre, stats, degt, gamma_b, beta_b, wlin_pad):
    R = 1000
    grid = (N // R,)
    return pl.pallas_call(
        _dense_b_body,
        grid=grid,
        in_specs=[
            pl.BlockSpec((R, H), lambda i: (i, 0)),
            pl.BlockSpec((8, H), lambda i: (0, 0)),
            pl.BlockSpec((R, NW), lambda i: (i, 0)),
            pl.BlockSpec((8, H), lambda i: (0, 0)),
            pl.BlockSpec((8, H), lambda i: (0, 0)),
            pl.BlockSpec((8, H), lambda i: (0, 0)),
        ],
        out_specs=pl.BlockSpec((1, 8, R), lambda i: (i, 0, 0)),
        out_shape=jax.ShapeDtypeStruct((N // R, 8, R), jnp.float32),
    )(hpre, stats, degt, gamma_b, beta_b, wlin_pad)


def _plane_slice(ref, j):
    return ref[j // 8, pl.ds((j % 8) * 16, 16)]


def _appnp_kernel(src_hbm, dst_hbm, planes_hbm, batch_hbm,
                  pool_out, cnt_out,
                  srcv, dstv, normv, zv, rv, accv, wv, tv, batchv,
                  rowidx, binp, binc, zaccsh, znewsh):
    c = lax.axis_index("c")
    s = lax.axis_index("s")

    pltpu.sync_copy(src_hbm.at[s], srcv)
    pltpu.sync_copy(dst_hbm.at[s], dstv)
    pltpu.sync_copy(planes_hbm.at[2], wv)
    pltpu.sync_copy(planes_hbm.at[c], rv)
    pltpu.sync_copy(planes_hbm.at[c], zv)
    pltpu.sync_copy(batch_hbm, batchv)

    for g in range(NR2 // 16):
        rowidx[pl.ds(g * 16, 16)] = lax.iota(jnp.int32, 16) + g * 16

    UN = 10
    def norm_body(j, _):
        sls = [pl.ds(j * (UN * 16) + u * 16, 16) for u in range(UN)]
        srs = [_rc(srcv[sl]) for sl in sls]
        drs = [_rc(dstv[sl]) for sl in sls]
        avs = [plsc.load_gather(wv, [r, c_]) for (r, c_) in srs]
        bvs = [plsc.load_gather(wv, [r, c_]) for (r, c_) in drs]
        for u in range(UN):
            normv[sls[u]] = avs[u] * bvs[u]
        return 0
    lax.fori_loop(0, EG2 // UN, norm_body, 0)

    pltpu.sync_copy(planes_hbm.at[3], wv)

    zeros16 = jnp.zeros((16,), jnp.float32)

    def zero_acc(j, _):
        for g in range(8):
            accv[j, pl.ds(g * 16, 16)] = zeros16
        return 0
    lax.fori_loop(0, NR2, zero_acc, 0)
    rb0 = s * TR2
    pltpu.sync_copy(accv.at[pl.ds(0, TR2)], zaccsh.at[pl.ds(rb0, TR2)])
    plsc.subcore_barrier()

    omal = jnp.float32(1.0 - ALPHA)
    al = jnp.float32(ALPHA)

    for _ in range(K):
        def edge_body(j, _):
            sls = [pl.ds(j * (UN * 16) + u * 16, 16) for u in range(UN)]
            srs = [_rc(srcv[sl]) for sl in sls]
            gs = [plsc.load_gather(zv, [r, c_]) for (r, c_) in srs]
            nos = [normv[sl] for sl in sls]
            drs = [_rc(dstv[sl]) for sl in sls]
            for u in range(UN):
                plsc.addupdate_scatter(accv, [drs[u][0], drs[u][1]],
                                       gs[u] * nos[u])
            return 0
        lax.fori_loop(0, EG2 // UN, edge_body, 0)

        pltpu.sync_copy(accv, zaccsh.at[rowidx], add=True)
        plsc.subcore_barrier()

        pltpu.sync_copy(zaccsh.at[pl.ds(rb0, TR2)], tv)

        lax.fori_loop(0, NR2, zero_acc, 0)
        pltpu.sync_copy(accv.at[pl.ds(0, TR2)], zaccsh.at[pl.ds(rb0, TR2)])

        def upd_body(j, _):
            jg = rb0 + j
            css = [pl.ds(g * 16, 16) for g in range(8)]
            tvs = [tv[j, cs] for cs in css]
            wvs = [wv[jg, cs] for cs in css]
            zvs = [zv[jg, cs] for cs in css]
            rvs = [rv[jg, cs] for cs in css]
            for g in range(8):
                tv[j, css[g]] = (omal * (tvs[g] + wvs[g] * zvs[g])
                                 + al * rvs[g])
            return 0
        lax.fori_loop(0, TR2, upd_body, 0)

        pltpu.sync_copy(tv, znewsh.at[pl.ds(rb0, TR2)])
        plsc.subcore_barrier()
        pltpu.sync_copy(znewsh, zv)

    @pl.when(s == 0)
    def _():
        for g in range(G // 16):
            binp[g, :] = jnp.zeros((16,), jnp.float32)
            binc[g, :] = jnp.zeros((16,), jnp.float32)

        ones16 = jnp.ones((16,), jnp.float32)

        def pool_body(j, _):
            jfs = [j * 5 + u for u in range(5)]
            bis = [batchv[pl.ds(jf * 16, 16)] for jf in jfs]
            brs = [lax.shift_right_logical(bi, 4) for bi in bis]
            bcs = [lax.bitwise_and(bi, 15) for bi in bis]
            zs = [_plane_slice(zv, jf) for jf in jfs]
            for u in range(5):
                plsc.addupdate_scatter(binp, [brs[u], bcs[u]], zs[u])
                plsc.addupdate_scatter(binc, [brs[u], bcs[u]], ones16)
            return 0
        lax.fori_loop(0, N // 16 // 5, pool_body, 0)

        pltpu.sync_copy(binp, pool_out.at[c])

        @pl.when(c == 0)
        def _():
            pltpu.sync_copy(binc, cnt_out)


def _appnp_sc(src2, dst2, planes, batch):
    mesh = plsc.VectorSubcoreMesh(core_axis_name="c", subcore_axis_name="s",
                                  num_cores=NC, num_subcores=NS)
    f = pl.kernel(
        _appnp_kernel,
        out_type=(jax.ShapeDtypeStruct((NC, G // 16, 16), jnp.float32),
                  jax.ShapeDtypeStruct((G // 16, 16), jnp.float32)),
        mesh=mesh,
        compiler_params=pltpu.CompilerParams(use_tc_tiling_on_sc=False, needs_layout_passes=False),
        scratch_types=[
            pltpu.VMEM((ET2,), jnp.int32),
            pltpu.VMEM((ET2,), jnp.int32),
            pltpu.VMEM((ET2,), jnp.float32),
            pltpu.VMEM((NR2, 128), jnp.float32),
            pltpu.VMEM((NR2, 128), jnp.float32),
            pltpu.VMEM((NR2, 128), jnp.float32),
            pltpu.VMEM((NR2, 128), jnp.float32),
            pltpu.VMEM((TR2, 128), jnp.float32),
            pltpu.VMEM((N,), jnp.int32),
            pltpu.VMEM((NR2,), jnp.int32),
            pltpu.VMEM((G // 16, 16), jnp.float32),
            pltpu.VMEM((G // 16, 16), jnp.float32),
            pltpu.VMEM_SHARED((NR2, 128), jnp.float32),
            pltpu.VMEM_SHARED((NR2, 128), jnp.float32),
        ],
    )
    return f(src2, dst2, planes, batch)


def kernel(x, edge_index, batch, W_l, W_r, b1, gamma, beta, W_lin, b_lin):
    src = edge_index[0]
    dst = edge_index[1]

    src3 = src.reshape(NW, NCH1, C1)
    dst3 = dst.reshape(NW, NCH1, C1)
    zeros_big = jnp.zeros((N, D), jnp.float32)

    aggp, degp = _sage_sc(x, src3, dst3, zeros_big)
    degt = degp.reshape(NW, N2)[:, :N].T

    b1_b = jnp.broadcast_to(b1[None, :], (8, H))
    hpre, stats = _dense_a(aggp, x, degt, W_l, W_r, b1_b)

    gamma_b = jnp.broadcast_to(gamma[None, :], (8, H))
    beta_b = jnp.broadcast_to(beta[None, :], (8, H))
    wlin_pad = jnp.zeros((8, H), jnp.float32).at[:OUT].set(W_lin)
    planes3 = _dense_b(hpre, stats, degt, gamma_b, beta_b, wlin_pad)

    planes = planes3.transpose(1, 0, 2).reshape(8, N)
    planes = jnp.pad(planes, ((0, 0), (0, N2 - N))).reshape(8, NR2, 128)

    src2 = src.reshape(NS, ET2)
    dst2 = dst.reshape(NS, ET2)
    pool, cnt = _appnp_sc(src2, dst2, planes, batch)
    pool = pool.reshape(NC, G)
    cnt = cnt.reshape(G)

    out = (pool / jnp.clip(cnt, 1.0, None)[None, :]).T + b_lin[None, :]
    return out


# speedup vs baseline: 106.3821x; 1.0186x over previous
import functools

import jax
import jax.numpy as jnp
from jax import lax
from jax.experimental import pallas as pl
from jax.experimental.pallas import tpu as pltpu
from jax.experimental.pallas import tpu_sc as plsc

N = 10000
E = 320000
D = 128
H = 128
OUT = 2
G = 64
K = 10
ALPHA = 0.1

NC = 2
NS = 16
NW = NC * NS

EW1 = E // NW
C1 = 80
NCH1 = EW1 // C1
NB1 = 3
DR = 80
NA = N + DR
NSL1 = NA // NS

ET2 = E // NS
EG2 = ET2 // 16
N2 = 10240
NSL2 = N2 // NS
NR2 = N2 // 128
TR2 = NR2 // NS
RB = 1000


def _rc(idx):
    return lax.shift_right_logical(idx, 7), lax.bitwise_and(idx, 127)


def _sage_kernel(x_hbm, edge_hbm,
                 aggp_out,
                 srcr, dstr, rows, degv, rowidx, aggsh, gsems, isems):
    c = lax.axis_index("c")
    s = lax.axis_index("s")
    wid = s * NC + c
    base = wid * EW1

    zeros16 = jnp.zeros((16,), jnp.float32)
    ones16 = jnp.ones((16,), jnp.float32)

    def zero_deg(j, _):
        for g in range(8):
            degv[j, pl.ds(g * 16, 16)] = zeros16
        return 0
    lax.fori_loop(0, NR2, zero_deg, 0)

    for g in range(DR // 16):
        rowidx[pl.ds(g * 16, 16)] = lax.iota(jnp.int32, 16) + (N + g * 16)

    for q in range(7):
        pltpu.sync_copy(degv, aggsh.at[pl.ds(s * NSL1 + q * 80, 80)])
    pltpu.sync_copy(degv.at[pl.ds(0, 70)],
                    aggsh.at[pl.ds(s * NSL1 + 560, 70)])

    plsc.subcore_barrier()

    for b in range(NB1):
        pltpu.async_copy(edge_hbm.at[0, pl.ds(base + b * C1, C1)],
                         srcr.at[b], isems.at[b])
        pltpu.async_copy(edge_hbm.at[1, pl.ds(base + b * C1, C1)],
                         dstr.at[b], isems.at[b])
    pltpu.make_async_copy(edge_hbm.at[0, pl.ds(base, C1)], srcr.at[0],
                          isems.at[0]).wait()
    pltpu.make_async_copy(edge_hbm.at[1, pl.ds(base, C1)], dstr.at[0],
                          isems.at[0]).wait()
    pltpu.async_copy(x_hbm.at[srcr.at[0]], rows.at[0], gsems.at[0])

    def do_chunk(j, b, bn):
        @pl.when(j + 1 < NCH1)
        def _():
            pltpu.make_async_copy(edge_hbm.at[0, pl.ds(base, C1)],
                                  srcr.at[bn], isems.at[bn]).wait()
            pltpu.make_async_copy(edge_hbm.at[1, pl.ds(base, C1)],
                                  dstr.at[bn], isems.at[bn]).wait()
            pltpu.async_copy(x_hbm.at[srcr.at[bn]], rows.at[bn],
                             gsems.at[bn])
        pltpu.make_async_copy(x_hbm.at[srcr.at[b]], rows.at[b],
                              gsems.at[b]).wait()
        pltpu.sync_copy(rows.at[b], aggsh.at[dstr.at[b]], add=True)
        sls = [pl.ds(g * 16, 16) for g in range(C1 // 16)]
        drcs = [_rc(dstr[b, sl]) for sl in sls]
        for (dr, dc) in drcs:
            plsc.addupdate_scatter(degv, [dr, dc], ones16)
        jn = j + NB1
        @pl.when(jn < NCH1)
        def _():
            pltpu.async_copy(edge_hbm.at[0, pl.ds(base + jn * C1, C1)],
                             srcr.at[b], isems.at[b])
            pltpu.async_copy(edge_hbm.at[1, pl.ds(base + jn * C1, C1)],
                             dstr.at[b], isems.at[b])

    def chunk_group(i, _):
        for b in range(NB1):
            do_chunk(i * NB1 + b, b, (b + 1) % NB1)
        return 0

    NFULL = NCH1 // NB1
    lax.fori_loop(0, NFULL, chunk_group, 0)
    for j in range(NFULL * NB1, NCH1):
        do_chunk(j, j % NB1, (j + 1) % NB1)

    pltpu.sync_copy(degv, aggsh.at[rowidx], add=True)

    plsc.subcore_barrier()

    pltpu.sync_copy(aggsh.at[pl.ds(s * NSL1, NSL1)],
                    aggp_out.at[c, pl.ds(s * NSL1, NSL1)])


def _sage_sc(x, edge_index):
    mesh = plsc.VectorSubcoreMesh(core_axis_name="c", subcore_axis_name="s",
                                  num_cores=NC, num_subcores=NS)
    f = pl.kernel(
        _sage_kernel,
        out_type=jax.ShapeDtypeStruct((NC, NA, D), jnp.float32),
        mesh=mesh,
        compiler_params=pltpu.CompilerParams(use_tc_tiling_on_sc=False,
                                             needs_layout_passes=False),
        scratch_types=[
            pltpu.VMEM((NB1, C1), jnp.int32),
            pltpu.VMEM((NB1, C1), jnp.int32),
            pltpu.VMEM((NB1, C1, D), jnp.float32),
            pltpu.VMEM((NR2, 128), jnp.float32),
            pltpu.VMEM((DR,), jnp.int32),
            pltpu.VMEM_SHARED((NA, D), jnp.float32),
            pltpu.SemaphoreType.DMA((NB1,)),
            pltpu.SemaphoreType.DMA((NB1,)),
        ],
    )
    return f(x, edge_index)


def _dense_a_body(aggp_ref, x_ref, degc_ref, wl_ref, wr_ref, b1_ref,
                  hpre_ref, stats_ref):
    i = pl.program_id(0)
    deg = degc_ref[...]
    agg = (aggp_ref[0] + aggp_ref[1]) / jnp.maximum(deg, 1.0)
    h = lax.dot_general(agg, wl_ref[...], (((1,), (1,)), ((), ())),
                        preferred_element_type=jnp.float32)
    h = h + lax.dot_general(x_ref[...], wr_ref[...], (((1,), (1,)), ((), ())),
                            preferred_element_type=jnp.float32)
    h = h + b1_ref[0:1, :]
    hpre_ref[...] = h
    bs = jnp.sum(h, axis=0, keepdims=True)
    bq = jnp.sum(h * h, axis=0, keepdims=True)
    upd = jnp.concatenate([bs, bq, jnp.zeros((6, 128), jnp.float32)], axis=0)

    @pl.when(i == 0)
    def _():
        stats_ref[...] = upd

    @pl.when(i > 0)
    def _():
        stats_ref[...] = stats_ref[...] + upd


def _dense_a(aggp, x, degc, W_l, W_r, b1_b):
    grid = (N // RB,)
    return pl.pallas_call(
        _dense_a_body,
        grid=grid,
        in_specs=[
            pl.BlockSpec((NC, RB, D), lambda i: (0, i, 0)),
            pl.BlockSpec((RB, D), lambda i: (i, 0)),
            pl.BlockSpec((RB, 1), lambda i: (i, 0)),
            pl.BlockSpec((H, D), lambda i: (0, 0)),
            pl.BlockSpec((H, D), lambda i: (0, 0)),
            pl.BlockSpec((8, H), lambda i: (0, 0)),
        ],
        out_specs=[
            pl.BlockSpec((RB, H), lambda i: (i, 0)),
            pl.BlockSpec((8, H), lambda i: (0, 0)),
        ],
        out_shape=[
            jax.ShapeDtypeStruct((N, H), jnp.float32),
            jax.ShapeDtypeStruct((8, H), jnp.float32),
        ],
    )(aggp, x, degc, W_l, W_r, b1_b)


def _dense_b_body(hpre_ref, stats_ref, degc_ref, gamma_ref, beta_ref,
                  wlin_ref, out_ref):
    stats = stats_ref[...]
    mean = stats[0:1, :] / N
    var = stats[1:2, :] / N - mean * mean
    inv = lax.rsqrt(var + 1e-5)
    hn = (hpre_ref[...] - mean) * (inv * gamma_ref[0:1, :]) + beta_ref[0:1, :]
    hn = jnp.maximum(hn, 0.0)
    r8 = lax.dot_general(wlin_ref[...], hn, (((1,), (1,)), ((), ())),
                         preferred_element_type=jnp.float32)
    deg = degc_ref[...][:, 0]
    d2 = 1.0 / (deg + 1.0)
    d1 = jnp.sqrt(d2)
    row = lax.broadcasted_iota(jnp.int32, r8.shape, 0)
    outb = jnp.where(row == 2, d1[None, :],
                     jnp.where(row == 3, d2[None, :], r8))
    out_ref[...] = outb.reshape(out_ref.shape)


def _dense_b(hpre, stats, degc, gamma_b, beta_b, wlin_pad):
    grid = (N // RB,)
    return pl.pallas_call(
        _dense_b_body,
        grid=grid,
        in_specs=[
            pl.BlockSpec((RB, H), lambda i: (i, 0)),
            pl.BlockSpec((8, H), lambda i: (0, 0)),
            pl.BlockSpec((RB, 1), lambda i: (i, 0)),
            pl.BlockSpec((8, H), lambda i: (0, 0)),
            pl.BlockSpec((8, H), lambda i: (0, 0)),
            pl.BlockSpec((8, H), lambda i: (0, 0)),
        ],
        out_specs=pl.BlockSpec((1, 8, RB), lambda i: (i, 0, 0)),
        out_shape=jax.ShapeDtypeStruct((N // RB, 8, RB), jnp.float32),
    )(hpre, stats, degc, gamma_b, beta_b, wlin_pad)


def _appnp_kernel(edge_hbm, planes_hbm, batch_hbm,
                  pool_out, cnt_out,
                  srcv, dstv, normv, zv, rv, accv, wv, tv, tnew, batchv,
                  rowidx, binp, binc, zaccsh, znewsh):
    c = lax.axis_index("c")
    s = lax.axis_index("s")
    ebase = s * ET2

    pltpu.sync_copy(edge_hbm.at[0, pl.ds(ebase, ET2)], srcv)
    pltpu.sync_copy(edge_hbm.at[1, pl.ds(ebase, ET2)], dstv)
    pltpu.sync_copy(batch_hbm, batchv)
    for i in range(N // RB):
        sl = pl.ds(i * RB, RB)
        pltpu.sync_copy(planes_hbm.at[i, 2], wv.at[sl])
        pltpu.sync_copy(planes_hbm.at[i, c], rv.at[sl])
    zeros16 = jnp.zeros((16,), jnp.float32)
    for g in range((N2 - N) // 16):
        wv[pl.ds(N + g * 16, 16)] = zeros16
        rv[pl.ds(N + g * 16, 16)] = zeros16
    def copy_z(j, _):
        for g in range(8):
            sl = pl.ds(j * 128 + g * 16, 16)
            zv[sl] = rv[sl]
        return 0
    lax.fori_loop(0, NR2, copy_z, 0)

    for g in range(NR2 // 16):
        rowidx[pl.ds(g * 16, 16)] = lax.iota(jnp.int32, 16) + g * 16

    UN = 10
    def norm_body(j, _):
        sls = [pl.ds(j * (UN * 16) + u * 16, 16) for u in range(UN)]
        avs = [plsc.load_gather(wv, [srcv[sl]]) for sl in sls]
        bvs = [plsc.load_gather(wv, [dstv[sl]]) for sl in sls]
        for u in range(UN):
            normv[sls[u]] = avs[u] * bvs[u]
        return 0
    lax.fori_loop(0, EG2 // UN, norm_body, 0)

    for i in range(N // RB):
        pltpu.sync_copy(planes_hbm.at[i, 3], wv.at[pl.ds(i * RB, RB)])

    def zero_acc(j, _):
        for g in range(8):
            accv[j, pl.ds(g * 16, 16)] = zeros16
        return 0
    lax.fori_loop(0, NR2, zero_acc, 0)
    rb0 = s * TR2
    pltpu.sync_copy(accv.at[pl.ds(0, TR2)], zaccsh.at[pl.ds(rb0, TR2)])
    plsc.subcore_barrier()

    omal = jnp.float32(1.0 - ALPHA)
    al = jnp.float32(ALPHA)
    nb0 = s * NSL2

    for _ in range(K):
        def edge_body(j, _):
            sls = [pl.ds(j * (UN * 16) + u * 16, 16) for u in range(UN)]
            gs = [plsc.load_gather(zv, [srcv[sl]]) for sl in sls]
            nos = [normv[sl] for sl in sls]
            drs = [_rc(dstv[sl]) for sl in sls]
            for u in range(UN):
                plsc.addupdate_scatter(accv, [drs[u][0], drs[u][1]],
                                       gs[u] * nos[u])
            return 0
        lax.fori_loop(0, EG2 // UN, edge_body, 0)

        pltpu.sync_copy(accv, zaccsh.at[rowidx], add=True)
        lax.fori_loop(0, NR2, zero_acc, 0)
        plsc.subcore_barrier()

        pltpu.sync_copy(zaccsh.at[pl.ds(rb0, TR2)], tv)
        pltpu.sync_copy(accv.at[pl.ds(0, TR2)], zaccsh.at[pl.ds(rb0, TR2)])

        def upd_body(j, _):
            css = [pl.ds(g * 16, 16) for g in range(8)]
            gsl = [pl.ds(nb0 + j * 128 + g * 16, 16) for g in range(8)]
            tvs = [tv[j, cs] for cs in css]
            wvs = [wv[sl] for sl in gsl]
            zvs = [zv[sl] for sl in gsl]
            rvs = [rv[sl] for sl in gsl]
            for g in range(8):
                tnew[pl.ds(j * 128 + g * 16, 16)] = (
                    omal * (tvs[g] + wvs[g] * zvs[g]) + al * rvs[g])
            return 0
        lax.fori_loop(0, TR2, upd_body, 0)

        pltpu.sync_copy(tnew, znewsh.at[pl.ds(nb0, NSL2)])
        plsc.subcore_barrier()
        pltpu.sync_copy(znewsh, zv)

    @pl.when(s == 0)
    def _():
        for g in range(G // 16):
            binp[g, :] = jnp.zeros((16,), jnp.float32)
            binc[g, :] = jnp.zeros((16,), jnp.float32)

        ones16 = jnp.ones((16,), jnp.float32)

        def pool_body(j, _):
            sls = [pl.ds((j * 5 + u) * 16, 16) for u in range(5)]
            bis = [batchv[sl] for sl in sls]
            brs = [lax.shift_right_logical(bi, 4) for bi in bis]
            bcs = [lax.bitwise_and(bi, 15) for bi in bis]
            zs = [zv[sl] for sl in sls]
            for u in range(5):
                plsc.addupdate_scatter(binp, [brs[u], bcs[u]], zs[u])
                plsc.addupdate_scatter(binc, [brs[u], bcs[u]], ones16)
            return 0
        lax.fori_loop(0, N // 16 // 5, pool_body, 0)

        pltpu.sync_copy(binp, pool_out.at[c])

        @pl.when(c == 0)
        def _():
            pltpu.sync_copy(binc, cnt_out)


def _appnp_sc(edge_index, planes3, batch):
    mesh = plsc.VectorSubcoreMesh(core_axis_name="c", subcore_axis_name="s",
                                  num_cores=NC, num_subcores=NS)
    f = pl.kernel(
        _appnp_kernel,
        out_type=(jax.ShapeDtypeStruct((NC, G // 16, 16), jnp.float32),
                  jax.ShapeDtypeStruct((G // 16, 16), jnp.float32)),
        mesh=mesh,
        compiler_params=pltpu.CompilerParams(use_tc_tiling_on_sc=False,
                                             needs_layout_passes=False),
        scratch_types=[
            pltpu.VMEM((ET2,), jnp.int32),
            pltpu.VMEM((ET2,), jnp.int32),
            pltpu.VMEM((ET2,), jnp.float32),
            pltpu.VMEM((N2,), jnp.float32),
            pltpu.VMEM((N2,), jnp.float32),
            pltpu.VMEM((NR2, 128), jnp.float32),
            pltpu.VMEM((N2,), jnp.float32),
            pltpu.VMEM((TR2, 128), jnp.float32),
            pltpu.VMEM((NSL2,), jnp.float32),
            pltpu.VMEM((N,), jnp.int32),
            pltpu.VMEM((NR2,), jnp.int32),
            pltpu.VMEM((G // 16, 16), jnp.float32),
            pltpu.VMEM((G // 16, 16), jnp.float32),
            pltpu.VMEM_SHARED((NR2, 128), jnp.float32),
            pltpu.VMEM_SHARED((N2,), jnp.float32),
        ],
    )
    return f(edge_index, planes3, batch)


def kernel(x, edge_index, batch, W_l, W_r, b1, gamma, beta, W_lin, b_lin):
    aggp = _sage_sc(x, edge_index)

    degc = (aggp[0, N:, :].reshape(DR * D)
            + aggp[1, N:, :].reshape(DR * D))[:N].reshape(N, 1)

    b1_b = jnp.broadcast_to(b1[None, :], (8, H))
    hpre, stats = _dense_a(aggp, x, degc, W_l, W_r, b1_b)

    gamma_b = jnp.broadcast_to(gamma[None, :], (8, H))
    beta_b = jnp.broadcast_to(beta[None, :], (8, H))
    wlin_pad = jnp.zeros((8, H), jnp.float32).at[:OUT].set(W_lin)
    planes3 = _dense_b(hpre, stats, degc, gamma_b, beta_b, wlin_pad)

    pool, cnt = _appnp_sc(edge_index, planes3, batch)
    pool = pool.reshape(NC, G)
    cnt = cnt.reshape(G)

    out = (pool / jnp.clip(cnt, 1.0, None)[None, :]).T + b_lin[None, :]
    return out


# speedup vs baseline: 112.0439x; 1.0532x over previous
import functools

import jax
import jax.numpy as jnp
from jax import lax
from jax.experimental import pallas as pl
from jax.experimental.pallas import tpu as pltpu
from jax.experimental.pallas import tpu_sc as plsc

N = 10000
E = 320000
D = 128
H = 128
OUT = 2
G = 64
K = 10
ALPHA = 0.1

NC = 2
NS = 16
NW = NC * NS

EW1 = E // NW
C1 = 80
NCH1 = EW1 // C1
NB1 = 3
DR = 80
NA = N + DR
NSL1 = NA // NS

ET2 = E // NS
EG2 = ET2 // 16
N2 = 10240
NSL2 = N2 // NS
NR2 = N2 // 128
TR2 = NR2 // NS
RB = 1000


def _rc(idx):
    return lax.shift_right_logical(idx, 7), lax.bitwise_and(idx, 127)


def _sage_kernel(x_hbm, edge_hbm,
                 aggp_out,
                 srcr, dstr, rows, degv, rowidx, aggsh, gsems, isems):
    c = lax.axis_index("c")
    s = lax.axis_index("s")
    wid = s * NC + c
    base = wid * EW1

    zeros16 = jnp.zeros((16,), jnp.float32)
    ones16 = jnp.ones((16,), jnp.float32)

    def zero_deg(j, _):
        for g in range(8):
            degv[j, pl.ds(g * 16, 16)] = zeros16
        return 0
    lax.fori_loop(0, NR2, zero_deg, 0)

    for g in range(DR // 16):
        rowidx[pl.ds(g * 16, 16)] = lax.iota(jnp.int32, 16) + (N + g * 16)

    for q in range(7):
        pltpu.sync_copy(degv, aggsh.at[pl.ds(s * NSL1 + q * 80, 80)])
    pltpu.sync_copy(degv.at[pl.ds(0, 70)],
                    aggsh.at[pl.ds(s * NSL1 + 560, 70)])

    plsc.subcore_barrier()

    for b in range(NB1):
        pltpu.async_copy(edge_hbm.at[0, pl.ds(base + b * C1, C1)],
                         srcr.at[b], isems.at[b])
        pltpu.async_copy(edge_hbm.at[1, pl.ds(base + b * C1, C1)],
                         dstr.at[b], isems.at[b])
    pltpu.make_async_copy(edge_hbm.at[0, pl.ds(base, C1)], srcr.at[0],
                          isems.at[0]).wait()
    pltpu.make_async_copy(edge_hbm.at[1, pl.ds(base, C1)], dstr.at[0],
                          isems.at[0]).wait()
    pltpu.async_copy(x_hbm.at[srcr.at[0]], rows.at[0], gsems.at[0])

    def do_chunk(j, b, bn):
        @pl.when(j + 1 < NCH1)
        def _():
            pltpu.make_async_copy(edge_hbm.at[0, pl.ds(base, C1)],
                                  srcr.at[bn], isems.at[bn]).wait()
            pltpu.make_async_copy(edge_hbm.at[1, pl.ds(base, C1)],
                                  dstr.at[bn], isems.at[bn]).wait()
            pltpu.async_copy(x_hbm.at[srcr.at[bn]], rows.at[bn],
                             gsems.at[bn])
        pltpu.make_async_copy(x_hbm.at[srcr.at[b]], rows.at[b],
                              gsems.at[b]).wait()
        pltpu.sync_copy(rows.at[b], aggsh.at[dstr.at[b]], add=True)
        sls = [pl.ds(g * 16, 16) for g in range(C1 // 16)]
        drcs = [_rc(dstr[b, sl]) for sl in sls]
        for (dr, dc) in drcs:
            plsc.addupdate_scatter(degv, [dr, dc], ones16)
        jn = j + NB1
        @pl.when(jn < NCH1)
        def _():
            pltpu.async_copy(edge_hbm.at[0, pl.ds(base + jn * C1, C1)],
                             srcr.at[b], isems.at[b])
            pltpu.async_copy(edge_hbm.at[1, pl.ds(base + jn * C1, C1)],
                             dstr.at[b], isems.at[b])

    def chunk_group(i, _):
        for b in range(NB1):
            do_chunk(i * NB1 + b, b, (b + 1) % NB1)
        return 0

    NFULL = NCH1 // NB1
    lax.fori_loop(0, NFULL, chunk_group, 0)
    for j in range(NFULL * NB1, NCH1):
        do_chunk(j, j % NB1, (j + 1) % NB1)

    pltpu.sync_copy(degv, aggsh.at[rowidx], add=True)

    plsc.subcore_barrier()

    pltpu.sync_copy(aggsh.at[pl.ds(s * NSL1, NSL1)],
                    aggp_out.at[c, pl.ds(s * NSL1, NSL1)])


def _sage_sc(x, edge_index):
    mesh = plsc.VectorSubcoreMesh(core_axis_name="c", subcore_axis_name="s",
                                  num_cores=NC, num_subcores=NS)
    f = pl.kernel(
        _sage_kernel,
        out_type=jax.ShapeDtypeStruct((NC, NA, D), jnp.float32),
        mesh=mesh,
        compiler_params=pltpu.CompilerParams(use_tc_tiling_on_sc=False,
                                             needs_layout_passes=False),
        scratch_types=[
            pltpu.VMEM((NB1, C1), jnp.int32),
            pltpu.VMEM((NB1, C1), jnp.int32),
            pltpu.VMEM((NB1, C1, D), jnp.float32),
            pltpu.VMEM((NR2, 128), jnp.float32),
            pltpu.VMEM((DR,), jnp.int32),
            pltpu.VMEM_SHARED((NA, D), jnp.float32),
            pltpu.SemaphoreType.DMA((NB1,)),
            pltpu.SemaphoreType.DMA((NB1,)),
        ],
    )
    return f(x, edge_index)


def _dense_a_body(aggp_ref, x_ref, degc_ref, wl_ref, wr_ref, b1_ref,
                  hpre_ref, stats_ref):
    i = pl.program_id(0)
    deg = degc_ref[...]
    agg = (aggp_ref[0] + aggp_ref[1]) / jnp.maximum(deg, 1.0)
    h = lax.dot_general(agg, wl_ref[...], (((1,), (1,)), ((), ())),
                        preferred_element_type=jnp.float32)
    h = h + lax.dot_general(x_ref[...], wr_ref[...], (((1,), (1,)), ((), ())),
                            preferred_element_type=jnp.float32)
    h = h + b1_ref[0:1, :]
    hpre_ref[...] = h
    bs = jnp.sum(h, axis=0, keepdims=True)
    bq = jnp.sum(h * h, axis=0, keepdims=True)
    upd = jnp.concatenate([bs, bq, jnp.zeros((6, 128), jnp.float32)], axis=0)

    @pl.when(i == 0)
    def _():
        stats_ref[...] = upd

    @pl.when(i > 0)
    def _():
        stats_ref[...] = stats_ref[...] + upd


def _dense_a(aggp, x, degc, W_l, W_r, b1_b):
    grid = (N // RB,)
    return pl.pallas_call(
        _dense_a_body,
        grid=grid,
        in_specs=[
            pl.BlockSpec((NC, RB, D), lambda i: (0, i, 0)),
            pl.BlockSpec((RB, D), lambda i: (i, 0)),
            pl.BlockSpec((RB, 1), lambda i: (i, 0)),
            pl.BlockSpec((H, D), lambda i: (0, 0)),
            pl.BlockSpec((H, D), lambda i: (0, 0)),
            pl.BlockSpec((8, H), lambda i: (0, 0)),
        ],
        out_specs=[
            pl.BlockSpec((RB, H), lambda i: (i, 0)),
            pl.BlockSpec((8, H), lambda i: (0, 0)),
        ],
        out_shape=[
            jax.ShapeDtypeStruct((N, H), jnp.float32),
            jax.ShapeDtypeStruct((8, H), jnp.float32),
        ],
    )(aggp, x, degc, W_l, W_r, b1_b)


def _dense_b_body(hpre_ref, stats_ref, degc_ref, gamma_ref, beta_ref,
                  wlin_ref, out_ref):
    stats = stats_ref[...]
    mean = stats[0:1, :] / N
    var = stats[1:2, :] / N - mean * mean
    inv = lax.rsqrt(var + 1e-5)
    hn = (hpre_ref[...] - mean) * (inv * gamma_ref[0:1, :]) + beta_ref[0:1, :]
    hn = jnp.maximum(hn, 0.0)
    r8 = lax.dot_general(wlin_ref[...], hn, (((1,), (1,)), ((), ())),
                         preferred_element_type=jnp.float32)
    deg = degc_ref[...][:, 0]
    d2 = 1.0 / (deg + 1.0)
    d1 = jnp.sqrt(d2)
    row = lax.broadcasted_iota(jnp.int32, r8.shape, 0)
    outb = jnp.where(row == 2, d1[None, :],
                     jnp.where(row == 3, d2[None, :], r8))
    out_ref[...] = outb.reshape(out_ref.shape)


def _dense_b(hpre, stats, degc, gamma_b, beta_b, wlin_pad):
    grid = (N // RB,)
    return pl.pallas_call(
        _dense_b_body,
        grid=grid,
        in_specs=[
            pl.BlockSpec((RB, H), lambda i: (i, 0)),
            pl.BlockSpec((8, H), lambda i: (0, 0)),
            pl.BlockSpec((RB, 1), lambda i: (i, 0)),
            pl.BlockSpec((8, H), lambda i: (0, 0)),
            pl.BlockSpec((8, H), lambda i: (0, 0)),
            pl.BlockSpec((8, H), lambda i: (0, 0)),
        ],
        out_specs=pl.BlockSpec((1, 8, RB), lambda i: (i, 0, 0)),
        out_shape=jax.ShapeDtypeStruct((N // RB, 8, RB), jnp.float32),
    )(hpre, stats, degc, gamma_b, beta_b, wlin_pad)


def _appnp_kernel(edge_hbm, planes_hbm, batch_hbm,
                  pool_out, cnt_out,
                  srcv, dstv, normv, zv, rv, accv, wv, tv, tnew, batchv,
                  rowidx, binp, binc, zaccsh, znewsh, psem):
    c = lax.axis_index("c")
    s = lax.axis_index("s")
    ebase = s * ET2

    pltpu.sync_copy(edge_hbm.at[0, pl.ds(ebase, ET2)], srcv)
    pltpu.sync_copy(edge_hbm.at[1, pl.ds(ebase, ET2)], dstv)
    pltpu.sync_copy(batch_hbm, batchv)
    descs = []
    for i in range(N // RB):
        sl = pl.ds(i * RB, RB)
        descs.append(pltpu.async_copy(planes_hbm.at[i, 2], wv.at[sl], psem))
        descs.append(pltpu.async_copy(planes_hbm.at[i, c], rv.at[sl], psem))
    for d in descs:
        d.wait()
    zeros16 = jnp.zeros((16,), jnp.float32)
    for g in range((N2 - N) // 16):
        wv[pl.ds(N + g * 16, 16)] = zeros16
        rv[pl.ds(N + g * 16, 16)] = zeros16
    def copy_z(j, _):
        for g in range(8):
            sl = pl.ds(j * 128 + g * 16, 16)
            zv[sl] = rv[sl]
        return 0
    lax.fori_loop(0, NR2, copy_z, 0)

    for g in range(NR2 // 16):
        rowidx[pl.ds(g * 16, 16)] = lax.iota(jnp.int32, 16) + g * 16

    UN = 10
    def norm_body(j, _):
        sls = [pl.ds(j * (UN * 16) + u * 16, 16) for u in range(UN)]
        avs = [plsc.load_gather(wv, [srcv[sl]]) for sl in sls]
        bvs = [plsc.load_gather(wv, [dstv[sl]]) for sl in sls]
        for u in range(UN):
            normv[sls[u]] = avs[u] * bvs[u]
        return 0
    lax.fori_loop(0, EG2 // UN, norm_body, 0)

    descs = [pltpu.async_copy(planes_hbm.at[i, 3], wv.at[pl.ds(i * RB, RB)],
                              psem)
             for i in range(N // RB)]
    for d in descs:
        d.wait()

    def zero_acc(j, _):
        for g in range(8):
            accv[j, pl.ds(g * 16, 16)] = zeros16
        return 0
    lax.fori_loop(0, NR2, zero_acc, 0)
    rb0 = s * TR2
    pltpu.sync_copy(accv.at[pl.ds(0, TR2)], zaccsh.at[pl.ds(rb0, TR2)])
    plsc.subcore_barrier()

    omal = jnp.float32(1.0 - ALPHA)
    al = jnp.float32(ALPHA)
    nb0 = s * NSL2

    for _ in range(K):
        def edge_body(j, _):
            sls = [pl.ds(j * (UN * 16) + u * 16, 16) for u in range(UN)]
            gs = [plsc.load_gather(zv, [srcv[sl]]) for sl in sls]
            nos = [normv[sl] for sl in sls]
            drs = [_rc(dstv[sl]) for sl in sls]
            for u in range(UN):
                plsc.addupdate_scatter(accv, [drs[u][0], drs[u][1]],
                                       gs[u] * nos[u])
            return 0
        lax.fori_loop(0, EG2 // UN, edge_body, 0)

        pltpu.sync_copy(accv, zaccsh.at[rowidx], add=True)
        lax.fori_loop(0, NR2, zero_acc, 0)
        plsc.subcore_barrier()

        pltpu.sync_copy(zaccsh.at[pl.ds(rb0, TR2)], tv)
        pltpu.sync_copy(accv.at[pl.ds(0, TR2)], zaccsh.at[pl.ds(rb0, TR2)])

        def upd_body(j, _):
            css = [pl.ds(g * 16, 16) for g in range(8)]
            gsl = [pl.ds(nb0 + j * 128 + g * 16, 16) for g in range(8)]
            tvs = [tv[j, cs] for cs in css]
            wvs = [wv[sl] for sl in gsl]
            zvs = [zv[sl] for sl in gsl]
            rvs = [rv[sl] for sl in gsl]
            for g in range(8):
                tnew[pl.ds(j * 128 + g * 16, 16)] = (
                    omal * (tvs[g] + wvs[g] * zvs[g]) + al * rvs[g])
            return 0
        lax.fori_loop(0, TR2, upd_body, 0)

        pltpu.sync_copy(tnew, znewsh.at[pl.ds(nb0, NSL2)])
        plsc.subcore_barrier()
        pltpu.sync_copy(znewsh, zv)

    @pl.when(s == 0)
    def _():
        for g in range(G // 16):
            binp[g, :] = jnp.zeros((16,), jnp.float32)
            binc[g, :] = jnp.zeros((16,), jnp.float32)

        ones16 = jnp.ones((16,), jnp.float32)

        def pool_body(j, _):
            sls = [pl.ds((j * 5 + u) * 16, 16) for u in range(5)]
            bis = [batchv[sl] for sl in sls]
            brs = [lax.shift_right_logical(bi, 4) for bi in bis]
            bcs = [lax.bitwise_and(bi, 15) for bi in bis]
            zs = [zv[sl] for sl in sls]
            for u in range(5):
                plsc.addupdate_scatter(binp, [brs[u], bcs[u]], zs[u])
                plsc.addupdate_scatter(binc, [brs[u], bcs[u]], ones16)
            return 0
        lax.fori_loop(0, N // 16 // 5, pool_body, 0)

        pltpu.sync_copy(binp, pool_out.at[c])

        @pl.when(c == 0)
        def _():
            pltpu.sync_copy(binc, cnt_out)


def _appnp_sc(edge_index, planes3, batch):
    mesh = plsc.VectorSubcoreMesh(core_axis_name="c", subcore_axis_name="s",
                                  num_cores=NC, num_subcores=NS)
    f = pl.kernel(
        _appnp_kernel,
        out_type=(jax.ShapeDtypeStruct((NC, G // 16, 16), jnp.float32),
                  jax.ShapeDtypeStruct((G // 16, 16), jnp.float32)),
        mesh=mesh,
        compiler_params=pltpu.CompilerParams(use_tc_tiling_on_sc=False,
                                             needs_layout_passes=False),
        scratch_types=[
            pltpu.VMEM((ET2,), jnp.int32),
            pltpu.VMEM((ET2,), jnp.int32),
            pltpu.VMEM((ET2,), jnp.float32),
            pltpu.VMEM((N2,), jnp.float32),
            pltpu.VMEM((N2,), jnp.float32),
            pltpu.VMEM((NR2, 128), jnp.float32),
            pltpu.VMEM((N2,), jnp.float32),
            pltpu.VMEM((TR2, 128), jnp.float32),
            pltpu.VMEM((NSL2,), jnp.float32),
            pltpu.VMEM((N,), jnp.int32),
            pltpu.VMEM((NR2,), jnp.int32),
            pltpu.VMEM((G // 16, 16), jnp.float32),
            pltpu.VMEM((G // 16, 16), jnp.float32),
            pltpu.VMEM_SHARED((NR2, 128), jnp.float32),
            pltpu.VMEM_SHARED((N2,), jnp.float32),
            pltpu.SemaphoreType.DMA,
        ],
    )
    return f(edge_index, planes3, batch)


def kernel(x, edge_index, batch, W_l, W_r, b1, gamma, beta, W_lin, b_lin):
    aggp = _sage_sc(x, edge_index)

    degc = (aggp[0, N:, :].reshape(DR * D)
            + aggp[1, N:, :].reshape(DR * D))[:N].reshape(N, 1)

    b1_b = jnp.broadcast_to(b1[None, :], (8, H))
    hpre, stats = _dense_a(aggp, x, degc, W_l, W_r, b1_b)

    gamma_b = jnp.broadcast_to(gamma[None, :], (8, H))
    beta_b = jnp.broadcast_to(beta[None, :], (8, H))
    wlin_pad = jnp.zeros((8, H), jnp.float32).at[:OUT].set(W_lin)
    planes3 = _dense_b(hpre, stats, degc, gamma_b, beta_b, wlin_pad)

    pool, cnt = _appnp_sc(edge_index, planes3, batch)
    pool = pool.reshape(NC, G)
    cnt = cnt.reshape(G)

    out = (pool / jnp.clip(cnt, 1.0, None)[None, :]).T + b_lin[None, :]
    return out


# speedup vs baseline: 119.2634x; 1.0644x over previous
import functools

import jax
import jax.numpy as jnp
from jax import lax
from jax.experimental import pallas as pl
from jax.experimental.pallas import tpu as pltpu
from jax.experimental.pallas import tpu_sc as plsc

N = 10000
E = 320000
D = 128
H = 128
OUT = 2
G = 64
K = 10
ALPHA = 0.1

NC = 2
NS = 16
NW = NC * NS

EW1 = E // NW
C1 = 80
NCH1 = EW1 // C1
NB1 = 3
NI1 = 6
DR = 80
NA = N + DR
NSL1 = NA // NS

ET2 = E // NS
EG2 = ET2 // 16
N2 = 10240
NSL2 = N2 // NS
NR2 = N2 // 128
TR2 = NR2 // NS
RB = 1000


def _rc(idx):
    return lax.shift_right_logical(idx, 7), lax.bitwise_and(idx, 127)


def _sage_kernel(x_hbm, edge_hbm,
                 aggp_out,
                 srcr, dstr, rows, degv, rowidx, aggsh, gsems, isems, ssems):
    c = lax.axis_index("c")
    s = lax.axis_index("s")
    wid = s * NC + c
    base = wid * EW1

    zeros16 = jnp.zeros((16,), jnp.float32)
    ones16 = jnp.ones((16,), jnp.float32)

    def zero_deg(j, _):
        for g in range(8):
            degv[j, pl.ds(g * 16, 16)] = zeros16
        return 0
    lax.fori_loop(0, NR2, zero_deg, 0)

    for g in range(DR // 16):
        rowidx[pl.ds(g * 16, 16)] = lax.iota(jnp.int32, 16) + (N + g * 16)

    for q in range(7):
        pltpu.sync_copy(degv, aggsh.at[pl.ds(s * NSL1 + q * 80, 80)])
    pltpu.sync_copy(degv.at[pl.ds(0, 70)],
                    aggsh.at[pl.ds(s * NSL1 + 560, 70)])

    plsc.subcore_barrier()

    for q in range(NI1):
        pltpu.async_copy(edge_hbm.at[0, pl.ds(base + q * C1, C1)],
                         srcr.at[q], isems.at[q])
        pltpu.async_copy(edge_hbm.at[1, pl.ds(base + q * C1, C1)],
                         dstr.at[q], isems.at[q])
    pltpu.make_async_copy(edge_hbm.at[0, pl.ds(base, C1)], srcr.at[0],
                          isems.at[0]).wait()
    pltpu.make_async_copy(edge_hbm.at[1, pl.ds(base, C1)], dstr.at[0],
                          isems.at[0]).wait()
    pltpu.async_copy(x_hbm.at[srcr.at[0]], rows.at[0], gsems.at[0])

    def do_chunk(j, b, bn, q, qn, drain=True):
        @pl.when(j + 1 < NCH1)
        def _():
            pltpu.make_async_copy(edge_hbm.at[0, pl.ds(base, C1)],
                                  srcr.at[qn], isems.at[qn]).wait()
            pltpu.make_async_copy(edge_hbm.at[1, pl.ds(base, C1)],
                                  dstr.at[qn], isems.at[qn]).wait()
            if drain:
                pltpu.make_async_copy(rows.at[bn], aggsh.at[dstr.at[qn]],
                                      ssems.at[bn]).wait()
            pltpu.async_copy(x_hbm.at[srcr.at[qn]], rows.at[bn],
                             gsems.at[bn])
        pltpu.make_async_copy(x_hbm.at[srcr.at[q]], rows.at[b],
                              gsems.at[b]).wait()
        pltpu.async_copy(rows.at[b], aggsh.at[dstr.at[q]], ssems.at[b],
                         add=True)
        sls = [pl.ds(g * 16, 16) for g in range(C1 // 16)]
        drcs = [_rc(dstr[q, sl]) for sl in sls]
        for (dr, dc) in drcs:
            plsc.addupdate_scatter(degv, [dr, dc], ones16)
        jn = j + NI1
        @pl.when(jn < NCH1)
        def _():
            pltpu.async_copy(edge_hbm.at[0, pl.ds(base + jn * C1, C1)],
                             srcr.at[q], isems.at[q])
            pltpu.async_copy(edge_hbm.at[1, pl.ds(base + jn * C1, C1)],
                             dstr.at[q], isems.at[q])

    for j in range(NI1):
        do_chunk(j, j % NB1, (j + 1) % NB1, j % NI1, (j + 1) % NI1,
                 drain=(j >= 2))

    def chunk_group(i, _):
        for u in range(NI1):
            j = i * NI1 + u
            do_chunk(j, u % NB1, (u + 1) % NB1, u, (u + 1) % NI1)
        return 0

    NFULL = NCH1 // NI1
    lax.fori_loop(1, NFULL, chunk_group, 0)
    for j in range(NFULL * NI1, NCH1):
        do_chunk(j, j % NB1, (j + 1) % NB1, j % NI1, (j + 1) % NI1)
    for b in range(NB1):
        pltpu.make_async_copy(rows.at[b], aggsh.at[dstr.at[0]],
                              ssems.at[b]).wait()

    pltpu.sync_copy(degv, aggsh.at[rowidx], add=True)

    plsc.subcore_barrier()

    pltpu.sync_copy(aggsh.at[pl.ds(s * NSL1, NSL1)],
                    aggp_out.at[c, pl.ds(s * NSL1, NSL1)])


def _sage_sc(x, edge_index):
    mesh = plsc.VectorSubcoreMesh(core_axis_name="c", subcore_axis_name="s",
                                  num_cores=NC, num_subcores=NS)
    f = pl.kernel(
        _sage_kernel,
        out_type=jax.ShapeDtypeStruct((NC, NA, D), jnp.float32),
        mesh=mesh,
        compiler_params=pltpu.CompilerParams(use_tc_tiling_on_sc=False,
                                             needs_layout_passes=False),
        scratch_types=[
            pltpu.VMEM((NI1, C1), jnp.int32),
            pltpu.VMEM((NI1, C1), jnp.int32),
            pltpu.VMEM((NB1, C1, D), jnp.float32),
            pltpu.VMEM((NR2, 128), jnp.float32),
            pltpu.VMEM((DR,), jnp.int32),
            pltpu.VMEM_SHARED((NA, D), jnp.float32),
            pltpu.SemaphoreType.DMA((NB1,)),
            pltpu.SemaphoreType.DMA((NI1,)),
            pltpu.SemaphoreType.DMA((NB1,)),
        ],
    )
    return f(x, edge_index)


def _dense_a_body(aggp_ref, x_ref, degc_ref, wl_ref, wr_ref, b1_ref,
                  hpre_ref, stats_ref):
    i = pl.program_id(0)
    deg = degc_ref[...]
    agg = (aggp_ref[0] + aggp_ref[1]) / jnp.maximum(deg, 1.0)
    h = lax.dot_general(agg, wl_ref[...], (((1,), (1,)), ((), ())),
                        preferred_element_type=jnp.float32)
    h = h + lax.dot_general(x_ref[...], wr_ref[...], (((1,), (1,)), ((), ())),
                            preferred_element_type=jnp.float32)
    h = h + b1_ref[0:1, :]
    hpre_ref[...] = h
    bs = jnp.sum(h, axis=0, keepdims=True)
    bq = jnp.sum(h * h, axis=0, keepdims=True)
    upd = jnp.concatenate([bs, bq, jnp.zeros((6, 128), jnp.float32)], axis=0)

    @pl.when(i == 0)
    def _():
        stats_ref[...] = upd

    @pl.when(i > 0)
    def _():
        stats_ref[...] = stats_ref[...] + upd


def _dense_a(aggp, x, degc, W_l, W_r, b1_b):
    grid = (N // RB,)
    return pl.pallas_call(
        _dense_a_body,
        grid=grid,
        in_specs=[
            pl.BlockSpec((NC, RB, D), lambda i: (0, i, 0)),
            pl.BlockSpec((RB, D), lambda i: (i, 0)),
            pl.BlockSpec((RB, 1), lambda i: (i, 0)),
            pl.BlockSpec((H, D), lambda i: (0, 0)),
            pl.BlockSpec((H, D), lambda i: (0, 0)),
            pl.BlockSpec((8, H), lambda i: (0, 0)),
        ],
        out_specs=[
            pl.BlockSpec((RB, H), lambda i: (i, 0)),
            pl.BlockSpec((8, H), lambda i: (0, 0)),
        ],
        out_shape=[
            jax.ShapeDtypeStruct((N, H), jnp.float32),
            jax.ShapeDtypeStruct((8, H), jnp.float32),
        ],
    )(aggp, x, degc, W_l, W_r, b1_b)


def _dense_b_body(hpre_ref, stats_ref, degc_ref, gamma_ref, beta_ref,
                  wlin_ref, out_ref):
    stats = stats_ref[...]
    mean = stats[0:1, :] / N
    var = stats[1:2, :] / N - mean * mean
    inv = lax.rsqrt(var + 1e-5)
    hn = (hpre_ref[...] - mean) * (inv * gamma_ref[0:1, :]) + beta_ref[0:1, :]
    hn = jnp.maximum(hn, 0.0)
    r8 = lax.dot_general(wlin_ref[...], hn, (((1,), (1,)), ((), ())),
                         preferred_element_type=jnp.float32)
    deg = degc_ref[...][:, 0]
    d2 = 1.0 / (deg + 1.0)
    d1 = jnp.sqrt(d2)
    row = lax.broadcasted_iota(jnp.int32, r8.shape, 0)
    outb = jnp.where(row == 2, d1[None, :],
                     jnp.where(row == 3, d2[None, :], r8))
    out_ref[...] = outb.reshape(out_ref.shape)


def _dense_b(hpre, stats, degc, gamma_b, beta_b, wlin_pad):
    grid = (N // RB,)
    return pl.pallas_call(
        _dense_b_body,
        grid=grid,
        in_specs=[
            pl.BlockSpec((RB, H), lambda i: (i, 0)),
            pl.BlockSpec((8, H), lambda i: (0, 0)),
            pl.BlockSpec((RB, 1), lambda i: (i, 0)),
            pl.BlockSpec((8, H), lambda i: (0, 0)),
            pl.BlockSpec((8, H), lambda i: (0, 0)),
            pl.BlockSpec((8, H), lambda i: (0, 0)),
        ],
        out_specs=pl.BlockSpec((1, 8, RB), lambda i: (i, 0, 0)),
        out_shape=jax.ShapeDtypeStruct((N // RB, 8, RB), jnp.float32),
    )(hpre, stats, degc, gamma_b, beta_b, wlin_pad)


def _appnp_kernel(edge_hbm, planes_hbm, batch_hbm,
                  pool_out, cnt_out,
                  srcv, dstv, normv, zv, rv, accv, wv, tv, tnew, batchv,
                  rowidx, binp, binc, zaccsh, znewsh, psem):
    c = lax.axis_index("c")
    s = lax.axis_index("s")
    ebase = s * ET2

    pltpu.sync_copy(edge_hbm.at[0, pl.ds(ebase, ET2)], srcv)
    pltpu.sync_copy(edge_hbm.at[1, pl.ds(ebase, ET2)], dstv)
    pltpu.sync_copy(batch_hbm, batchv)
    descs = []
    for i in range(N // RB):
        sl = pl.ds(i * RB, RB)
        descs.append(pltpu.async_copy(planes_hbm.at[i, 2], wv.at[sl], psem))
        descs.append(pltpu.async_copy(planes_hbm.at[i, c], rv.at[sl], psem))
    for d in descs:
        d.wait()
    zeros16 = jnp.zeros((16,), jnp.float32)
    for g in range((N2 - N) // 16):
        wv[pl.ds(N + g * 16, 16)] = zeros16
        rv[pl.ds(N + g * 16, 16)] = zeros16
    def copy_z(j, _):
        for g in range(8):
            sl = pl.ds(j * 128 + g * 16, 16)
            zv[sl] = rv[sl]
        return 0
    lax.fori_loop(0, NR2, copy_z, 0)

    for g in range(NR2 // 16):
        rowidx[pl.ds(g * 16, 16)] = lax.iota(jnp.int32, 16) + g * 16

    UN = 10
    def norm_body(j, _):
        sls = [pl.ds(j * (UN * 16) + u * 16, 16) for u in range(UN)]
        avs = [plsc.load_gather(wv, [srcv[sl]]) for sl in sls]
        bvs = [plsc.load_gather(wv, [dstv[sl]]) for sl in sls]
        for u in range(UN):
            normv[sls[u]] = avs[u] * bvs[u]
        return 0
    lax.fori_loop(0, EG2 // UN, norm_body, 0)

    descs = [pltpu.async_copy(planes_hbm.at[i, 3], wv.at[pl.ds(i * RB, RB)],
                              psem)
             for i in range(N // RB)]
    for d in descs:
        d.wait()

    def zero_acc(j, _):
        for g in range(8):
            accv[j, pl.ds(g * 16, 16)] = zeros16
        return 0
    lax.fori_loop(0, NR2, zero_acc, 0)
    rb0 = s * TR2
    pltpu.sync_copy(accv.at[pl.ds(0, TR2)], zaccsh.at[pl.ds(rb0, TR2)])
    plsc.subcore_barrier()

    omal = jnp.float32(1.0 - ALPHA)
    al = jnp.float32(ALPHA)
    nb0 = s * NSL2

    for _ in range(K):
        def edge_body(j, _):
            sls = [pl.ds(j * (UN * 16) + u * 16, 16) for u in range(UN)]
            gs = [plsc.load_gather(zv, [srcv[sl]]) for sl in sls]
            nos = [normv[sl] for sl in sls]
            drs = [_rc(dstv[sl]) for sl in sls]
            for u in range(UN):
                plsc.addupdate_scatter(accv, [drs[u][0], drs[u][1]],
                                       gs[u] * nos[u])
            return 0
        lax.fori_loop(0, EG2 // UN, edge_body, 0)

        pltpu.sync_copy(accv, zaccsh.at[rowidx], add=True)
        lax.fori_loop(0, NR2, zero_acc, 0)
        plsc.subcore_barrier()

        pltpu.sync_copy(zaccsh.at[pl.ds(rb0, TR2)], tv)
        pltpu.sync_copy(accv.at[pl.ds(0, TR2)], zaccsh.at[pl.ds(rb0, TR2)])

        def upd_body(j, _):
            css = [pl.ds(g * 16, 16) for g in range(8)]
            gsl = [pl.ds(nb0 + j * 128 + g * 16, 16) for g in range(8)]
            tvs = [tv[j, cs] for cs in css]
            wvs = [wv[sl] for sl in gsl]
            zvs = [zv[sl] for sl in gsl]
            rvs = [rv[sl] for sl in gsl]
            for g in range(8):
                tnew[pl.ds(j * 128 + g * 16, 16)] = (
                    omal * (tvs[g] + wvs[g] * zvs[g]) + al * rvs[g])
            return 0
        lax.fori_loop(0, TR2, upd_body, 0)

        pltpu.sync_copy(tnew, znewsh.at[pl.ds(nb0, NSL2)])
        plsc.subcore_barrier()
        pltpu.sync_copy(znewsh, zv)

    @pl.when(s == 0)
    def _():
        for g in range(G // 16):
            binp[g, :] = jnp.zeros((16,), jnp.float32)
            binc[g, :] = jnp.zeros((16,), jnp.float32)

        ones16 = jnp.ones((16,), jnp.float32)

        def pool_body(j, _):
            sls = [pl.ds((j * 5 + u) * 16, 16) for u in range(5)]
            bis = [batchv[sl] for sl in sls]
            brs = [lax.shift_right_logical(bi, 4) for bi in bis]
            bcs = [lax.bitwise_and(bi, 15) for bi in bis]
            zs = [zv[sl] for sl in sls]
            for u in range(5):
                plsc.addupdate_scatter(binp, [brs[u], bcs[u]], zs[u])
                plsc.addupdate_scatter(binc, [brs[u], bcs[u]], ones16)
            return 0
        lax.fori_loop(0, N // 16 // 5, pool_body, 0)

        pltpu.sync_copy(binp, pool_out.at[c])

        @pl.when(c == 0)
        def _():
            pltpu.sync_copy(binc, cnt_out)


def _appnp_sc(edge_index, planes3, batch):
    mesh = plsc.VectorSubcoreMesh(core_axis_name="c", subcore_axis_name="s",
                                  num_cores=NC, num_subcores=NS)
    f = pl.kernel(
        _appnp_kernel,
        out_type=(jax.ShapeDtypeStruct((NC, G // 16, 16), jnp.float32),
                  jax.ShapeDtypeStruct((G // 16, 16), jnp.float32)),
        mesh=mesh,
        compiler_params=pltpu.CompilerParams(use_tc_tiling_on_sc=False,
                                             needs_layout_passes=False),
        scratch_types=[
            pltpu.VMEM((ET2,), jnp.int32),
            pltpu.VMEM((ET2,), jnp.int32),
            pltpu.VMEM((ET2,), jnp.float32),
            pltpu.VMEM((N2,), jnp.float32),
            pltpu.VMEM((N2,), jnp.float32),
            pltpu.VMEM((NR2, 128), jnp.float32),
            pltpu.VMEM((N2,), jnp.float32),
            pltpu.VMEM((TR2, 128), jnp.float32),
            pltpu.VMEM((NSL2,), jnp.float32),
            pltpu.VMEM((N,), jnp.int32),
            pltpu.VMEM((NR2,), jnp.int32),
            pltpu.VMEM((G // 16, 16), jnp.float32),
            pltpu.VMEM((G // 16, 16), jnp.float32),
            pltpu.VMEM_SHARED((NR2, 128), jnp.float32),
            pltpu.VMEM_SHARED((N2,), jnp.float32),
            pltpu.SemaphoreType.DMA,
        ],
    )
    return f(edge_index, planes3, batch)


def kernel(x, edge_index, batch, W_l, W_r, b1, gamma, beta, W_lin, b_lin):
    aggp = _sage_sc(x, edge_index)

    degc = (aggp[0, N:, :].reshape(DR * D)
            + aggp[1, N:, :].reshape(DR * D))[:N].reshape(N, 1)

    b1_b = jnp.broadcast_to(b1[None, :], (8, H))
    hpre, stats = _dense_a(aggp, x, degc, W_l, W_r, b1_b)

    gamma_b = jnp.broadcast_to(gamma[None, :], (8, H))
    beta_b = jnp.broadcast_to(beta[None, :], (8, H))
    wlin_pad = jnp.zeros((8, H), jnp.float32).at[:OUT].set(W_lin)
    planes3 = _dense_b(hpre, stats, degc, gamma_b, beta_b, wlin_pad)

    pool, cnt = _appnp_sc(edge_index, planes3, batch)
    pool = pool.reshape(NC, G)
    cnt = cnt.reshape(G)

    out = (pool / jnp.clip(cnt, 1.0, None)[None, :]).T + b_lin[None, :]
    return out


# speedup vs baseline: 119.4492x; 1.0016x over previous
import functools

import jax
import jax.numpy as jnp
from jax import lax
from jax.experimental import pallas as pl
from jax.experimental.pallas import tpu as pltpu
from jax.experimental.pallas import tpu_sc as plsc

N = 10000
E = 320000
D = 128
H = 128
OUT = 2
G = 64
K = 10
ALPHA = 0.1

NC = 2
NS = 16
NW = NC * NS

EW1 = E // NW
C1 = 80
NCH1 = EW1 // C1
NB1 = 3
NI1 = 6
DR = 80
NA = N + DR
NSL1 = NA // NS

ET2 = E // NS
EG2 = ET2 // 16
N2 = 10240
NSL2 = N2 // NS
NR2 = N2 // 128
TR2 = NR2 // NS
RB = 1000


def _rc(idx):
    return lax.shift_right_logical(idx, 7), lax.bitwise_and(idx, 127)


def _sage_kernel(x_hbm, edge_hbm,
                 aggp_out,
                 srcr, dstr, rows, degv, rowidx, aggsh, gsems, isems, ssems):
    c = lax.axis_index("c")
    s = lax.axis_index("s")
    wid = s * NC + c
    base = wid * EW1

    zeros16 = jnp.zeros((16,), jnp.float32)
    ones16 = jnp.ones((16,), jnp.float32)

    def zero_deg(j, _):
        for g in range(8):
            degv[j, pl.ds(g * 16, 16)] = zeros16
        return 0
    lax.fori_loop(0, NR2, zero_deg, 0)

    for g in range(DR // 16):
        rowidx[pl.ds(g * 16, 16)] = lax.iota(jnp.int32, 16) + (N + g * 16)

    for q in range(7):
        pltpu.sync_copy(degv, aggsh.at[pl.ds(s * NSL1 + q * 80, 80)])
    pltpu.sync_copy(degv.at[pl.ds(0, 70)],
                    aggsh.at[pl.ds(s * NSL1 + 560, 70)])

    plsc.subcore_barrier()

    for q in range(NI1):
        pltpu.async_copy(edge_hbm.at[0, pl.ds(base + q * C1, C1)],
                         srcr.at[q], isems.at[q])
        pltpu.async_copy(edge_hbm.at[1, pl.ds(base + q * C1, C1)],
                         dstr.at[q], isems.at[q])
    pltpu.make_async_copy(edge_hbm.at[0, pl.ds(base, C1)], srcr.at[0],
                          isems.at[0]).wait()
    pltpu.make_async_copy(edge_hbm.at[1, pl.ds(base, C1)], dstr.at[0],
                          isems.at[0]).wait()
    pltpu.async_copy(x_hbm.at[srcr.at[0]], rows.at[0], gsems.at[0])

    def do_chunk(j, b, bn, q, qn, drain=True):
        @pl.when(j + 1 < NCH1)
        def _():
            pltpu.make_async_copy(edge_hbm.at[0, pl.ds(base, C1)],
                                  srcr.at[qn], isems.at[qn]).wait()
            pltpu.make_async_copy(edge_hbm.at[1, pl.ds(base, C1)],
                                  dstr.at[qn], isems.at[qn]).wait()
            if drain:
                pltpu.make_async_copy(rows.at[bn], aggsh.at[dstr.at[qn]],
                                      ssems.at[bn]).wait()
            pltpu.async_copy(x_hbm.at[srcr.at[qn]], rows.at[bn],
                             gsems.at[bn])
        pltpu.make_async_copy(x_hbm.at[srcr.at[q]], rows.at[b],
                              gsems.at[b]).wait()
        pltpu.async_copy(rows.at[b], aggsh.at[dstr.at[q]], ssems.at[b],
                         add=True)
        sls = [pl.ds(g * 16, 16) for g in range(C1 // 16)]
        drcs = [_rc(dstr[q, sl]) for sl in sls]
        for (dr, dc) in drcs:
            plsc.addupdate_scatter(degv, [dr, dc], ones16)
        jn = j + NI1
        @pl.when(jn < NCH1)
        def _():
            pltpu.async_copy(edge_hbm.at[0, pl.ds(base + jn * C1, C1)],
                             srcr.at[q], isems.at[q])
            pltpu.async_copy(edge_hbm.at[1, pl.ds(base + jn * C1, C1)],
                             dstr.at[q], isems.at[q])

    for j in range(NI1):
        do_chunk(j, j % NB1, (j + 1) % NB1, j % NI1, (j + 1) % NI1,
                 drain=(j >= 2))

    def chunk_group(i, _):
        for u in range(NI1):
            j = i * NI1 + u
            do_chunk(j, u % NB1, (u + 1) % NB1, u, (u + 1) % NI1)
        return 0

    NFULL = NCH1 // NI1
    lax.fori_loop(1, NFULL, chunk_group, 0)
    for j in range(NFULL * NI1, NCH1):
        do_chunk(j, j % NB1, (j + 1) % NB1, j % NI1, (j + 1) % NI1)
    for b in range(NB1):
        pltpu.make_async_copy(rows.at[b], aggsh.at[dstr.at[0]],
                              ssems.at[b]).wait()

    pltpu.sync_copy(degv, aggsh.at[rowidx], add=True)

    plsc.subcore_barrier()

    pltpu.sync_copy(aggsh.at[pl.ds(s * NSL1, NSL1)],
                    aggp_out.at[c, pl.ds(s * NSL1, NSL1)])


def _sage_sc(x, edge_index):
    mesh = plsc.VectorSubcoreMesh(core_axis_name="c", subcore_axis_name="s",
                                  num_cores=NC, num_subcores=NS)
    f = pl.kernel(
        _sage_kernel,
        out_type=jax.ShapeDtypeStruct((NC, NA, D), jnp.float32),
        mesh=mesh,
        compiler_params=pltpu.CompilerParams(use_tc_tiling_on_sc=False,
                                             needs_layout_passes=False),
        scratch_types=[
            pltpu.VMEM((NI1, C1), jnp.int32),
            pltpu.VMEM((NI1, C1), jnp.int32),
            pltpu.VMEM((NB1, C1, D), jnp.float32),
            pltpu.VMEM((NR2, 128), jnp.float32),
            pltpu.VMEM((DR,), jnp.int32),
            pltpu.VMEM_SHARED((NA, D), jnp.float32),
            pltpu.SemaphoreType.DMA((NB1,)),
            pltpu.SemaphoreType.DMA((NI1,)),
            pltpu.SemaphoreType.DMA((NB1,)),
        ],
    )
    return f(x, edge_index)


def _dense_xr_body(x_ref, wr_ref, b1_ref, xr_ref):
    xr_ref[...] = lax.dot_general(
        x_ref[...], wr_ref[...], (((1,), (1,)), ((), ())),
        preferred_element_type=jnp.float32) + b1_ref[0:1, :]


def _dense_xr(x, W_r, b1_b):
    return pl.pallas_call(
        _dense_xr_body,
        grid=(N // RB,),
        in_specs=[
            pl.BlockSpec((RB, D), lambda i: (i, 0)),
            pl.BlockSpec((H, D), lambda i: (0, 0)),
            pl.BlockSpec((8, H), lambda i: (0, 0)),
        ],
        out_specs=pl.BlockSpec((RB, H), lambda i: (i, 0)),
        out_shape=jax.ShapeDtypeStruct((N, H), jnp.float32),
    )(x, W_r, b1_b)


def _dense_a_body(aggp_ref, xr_ref, degc_ref, wl_ref,
                  hpre_ref, stats_ref):
    i = pl.program_id(0)
    deg = degc_ref[...]
    agg = (aggp_ref[0] + aggp_ref[1]) / jnp.maximum(deg, 1.0)
    h = lax.dot_general(agg, wl_ref[...], (((1,), (1,)), ((), ())),
                        preferred_element_type=jnp.float32)
    h = h + xr_ref[...]
    hpre_ref[...] = h
    bs = jnp.sum(h, axis=0, keepdims=True)
    bq = jnp.sum(h * h, axis=0, keepdims=True)
    upd = jnp.concatenate([bs, bq, jnp.zeros((6, 128), jnp.float32)], axis=0)

    @pl.when(i == 0)
    def _():
        stats_ref[...] = upd

    @pl.when(i > 0)
    def _():
        stats_ref[...] = stats_ref[...] + upd


def _dense_a(aggp, xr, degc, W_l):
    grid = (N // RB,)
    return pl.pallas_call(
        _dense_a_body,
        grid=grid,
        in_specs=[
            pl.BlockSpec((NC, RB, D), lambda i: (0, i, 0)),
            pl.BlockSpec((RB, H), lambda i: (i, 0)),
            pl.BlockSpec((RB, 1), lambda i: (i, 0)),
            pl.BlockSpec((H, D), lambda i: (0, 0)),
        ],
        out_specs=[
            pl.BlockSpec((RB, H), lambda i: (i, 0)),
            pl.BlockSpec((8, H), lambda i: (0, 0)),
        ],
        out_shape=[
            jax.ShapeDtypeStruct((N, H), jnp.float32),
            jax.ShapeDtypeStruct((8, H), jnp.float32),
        ],
    )(aggp, xr, degc, W_l)


def _dense_b_body(hpre_ref, stats_ref, degc_ref, gamma_ref, beta_ref,
                  wlin_ref, out_ref):
    stats = stats_ref[...]
    mean = stats[0:1, :] / N
    var = stats[1:2, :] / N - mean * mean
    inv = lax.rsqrt(var + 1e-5)
    hn = (hpre_ref[...] - mean) * (inv * gamma_ref[0:1, :]) + beta_ref[0:1, :]
    hn = jnp.maximum(hn, 0.0)
    r8 = lax.dot_general(wlin_ref[...], hn, (((1,), (1,)), ((), ())),
                         preferred_element_type=jnp.float32)
    deg = degc_ref[...][:, 0]
    d2 = 1.0 / (deg + 1.0)
    d1 = jnp.sqrt(d2)
    row = lax.broadcasted_iota(jnp.int32, r8.shape, 0)
    outb = jnp.where(row == 2, d1[None, :],
                     jnp.where(row == 3, d2[None, :], r8))
    out_ref[...] = outb.reshape(out_ref.shape)


def _dense_b(hpre, stats, degc, gamma_b, beta_b, wlin_pad):
    grid = (N // RB,)
    return pl.pallas_call(
        _dense_b_body,
        grid=grid,
        in_specs=[
            pl.BlockSpec((RB, H), lambda i: (i, 0)),
            pl.BlockSpec((8, H), lambda i: (0, 0)),
            pl.BlockSpec((RB, 1), lambda i: (i, 0)),
            pl.BlockSpec((8, H), lambda i: (0, 0)),
            pl.BlockSpec((8, H), lambda i: (0, 0)),
            pl.BlockSpec((8, H), lambda i: (0, 0)),
        ],
        out_specs=pl.BlockSpec((1, 8, RB), lambda i: (i, 0, 0)),
        out_shape=jax.ShapeDtypeStruct((N // RB, 8, RB), jnp.float32),
    )(hpre, stats, degc, gamma_b, beta_b, wlin_pad)


def _appnp_kernel(edge_hbm, planes_hbm, batch_hbm,
                  pool_out, cnt_out,
                  srcv, dstv, normv, zv, rv, accv, wv, tv, tnew, batchv,
                  rowidx, binp, binc, zaccsh, znewsh, psem):
    c = lax.axis_index("c")
    s = lax.axis_index("s")
    ebase = s * ET2

    pltpu.sync_copy(edge_hbm.at[0, pl.ds(ebase, ET2)], srcv)
    pltpu.sync_copy(edge_hbm.at[1, pl.ds(ebase, ET2)], dstv)
    pltpu.sync_copy(batch_hbm, batchv)
    descs = []
    for i in range(N // RB):
        sl = pl.ds(i * RB, RB)
        descs.append(pltpu.async_copy(planes_hbm.at[i, 2], wv.at[sl], psem))
        descs.append(pltpu.async_copy(planes_hbm.at[i, c], rv.at[sl], psem))
    for d in descs:
        d.wait()
    zeros16 = jnp.zeros((16,), jnp.float32)
    for g in range((N2 - N) // 16):
        wv[pl.ds(N + g * 16, 16)] = zeros16
        rv[pl.ds(N + g * 16, 16)] = zeros16
    def copy_z(j, _):
        for g in range(8):
            sl = pl.ds(j * 128 + g * 16, 16)
            zv[sl] = rv[sl]
        return 0
    lax.fori_loop(0, NR2, copy_z, 0)

    for g in range(NR2 // 16):
        rowidx[pl.ds(g * 16, 16)] = lax.iota(jnp.int32, 16) + g * 16

    UN = 10
    def norm_body(j, _):
        sls = [pl.ds(j * (UN * 16) + u * 16, 16) for u in range(UN)]
        avs = [plsc.load_gather(wv, [srcv[sl]]) for sl in sls]
        bvs = [plsc.load_gather(wv, [dstv[sl]]) for sl in sls]
        for u in range(UN):
            normv[sls[u]] = avs[u] * bvs[u]
        return 0
    lax.fori_loop(0, EG2 // UN, norm_body, 0)

    descs = [pltpu.async_copy(planes_hbm.at[i, 3], wv.at[pl.ds(i * RB, RB)],
                              psem)
             for i in range(N // RB)]
    for d in descs:
        d.wait()

    def zero_acc(j, _):
        for g in range(8):
            accv[j, pl.ds(g * 16, 16)] = zeros16
        return 0
    lax.fori_loop(0, NR2, zero_acc, 0)
    rb0 = s * TR2
    pltpu.sync_copy(accv.at[pl.ds(0, TR2)], zaccsh.at[pl.ds(rb0, TR2)])
    plsc.subcore_barrier()

    omal = jnp.float32(1.0 - ALPHA)
    al = jnp.float32(ALPHA)
    nb0 = s * NSL2

    for _ in range(K):
        def edge_body(j, _):
            sls = [pl.ds(j * (UN * 16) + u * 16, 16) for u in range(UN)]
            gs = [plsc.load_gather(zv, [srcv[sl]]) for sl in sls]
            nos = [normv[sl] for sl in sls]
            drs = [_rc(dstv[sl]) for sl in sls]
            for u in range(UN):
                plsc.addupdate_scatter(accv, [drs[u][0], drs[u][1]],
                                       gs[u] * nos[u])
            return 0
        lax.fori_loop(0, EG2 // UN, edge_body, 0)

        pltpu.sync_copy(accv, zaccsh.at[rowidx], add=True)
        lax.fori_loop(0, NR2, zero_acc, 0)
        plsc.subcore_barrier()

        pltpu.sync_copy(zaccsh.at[pl.ds(rb0, TR2)], tv)
        pltpu.sync_copy(accv.at[pl.ds(0, TR2)], zaccsh.at[pl.ds(rb0, TR2)])

        def upd_body(j, _):
            css = [pl.ds(g * 16, 16) for g in range(8)]
            gsl = [pl.ds(nb0 + j * 128 + g * 16, 16) for g in range(8)]
            tvs = [tv[j, cs] for cs in css]
            wvs = [wv[sl] for sl in gsl]
            zvs = [zv[sl] for sl in gsl]
            rvs = [rv[sl] for sl in gsl]
            for g in range(8):
                tnew[pl.ds(j * 128 + g * 16, 16)] = (
                    omal * (tvs[g] + wvs[g] * zvs[g]) + al * rvs[g])
            return 0
        lax.fori_loop(0, TR2, upd_body, 0)

        pltpu.sync_copy(tnew, znewsh.at[pl.ds(nb0, NSL2)])
        plsc.subcore_barrier()
        pltpu.sync_copy(znewsh, zv)

    @pl.when(s == 0)
    def _():
        for g in range(G // 16):
            binp[g, :] = jnp.zeros((16,), jnp.float32)
            binc[g, :] = jnp.zeros((16,), jnp.float32)

        ones16 = jnp.ones((16,), jnp.float32)

        def pool_body(j, _):
            sls = [pl.ds((j * 5 + u) * 16, 16) for u in range(5)]
            bis = [batchv[sl] for sl in sls]
            brs = [lax.shift_right_logical(bi, 4) for bi in bis]
            bcs = [lax.bitwise_and(bi, 15) for bi in bis]
            zs = [zv[sl] for sl in sls]
            for u in range(5):
                plsc.addupdate_scatter(binp, [brs[u], bcs[u]], zs[u])
                plsc.addupdate_scatter(binc, [brs[u], bcs[u]], ones16)
            return 0
        lax.fori_loop(0, N // 16 // 5, pool_body, 0)

        pltpu.sync_copy(binp, pool_out.at[c])

        @pl.when(c == 0)
        def _():
            pltpu.sync_copy(binc, cnt_out)


def _appnp_sc(edge_index, planes3, batch):
    mesh = plsc.VectorSubcoreMesh(core_axis_name="c", subcore_axis_name="s",
                                  num_cores=NC, num_subcores=NS)
    f = pl.kernel(
        _appnp_kernel,
        out_type=(jax.ShapeDtypeStruct((NC, G // 16, 16), jnp.float32),
                  jax.ShapeDtypeStruct((G // 16, 16), jnp.float32)),
        mesh=mesh,
        compiler_params=pltpu.CompilerParams(use_tc_tiling_on_sc=False,
                                             needs_layout_passes=False),
        scratch_types=[
            pltpu.VMEM((ET2,), jnp.int32),
            pltpu.VMEM((ET2,), jnp.int32),
            pltpu.VMEM((ET2,), jnp.float32),
            pltpu.VMEM((N2,), jnp.float32),
            pltpu.VMEM((N2,), jnp.float32),
            pltpu.VMEM((NR2, 128), jnp.float32),
            pltpu.VMEM((N2,), jnp.float32),
            pltpu.VMEM((TR2, 128), jnp.float32),
            pltpu.VMEM((NSL2,), jnp.float32),
            pltpu.VMEM((N,), jnp.int32),
            pltpu.VMEM((NR2,), jnp.int32),
            pltpu.VMEM((G // 16, 16), jnp.float32),
            pltpu.VMEM((G // 16, 16), jnp.float32),
            pltpu.VMEM_SHARED((NR2, 128), jnp.float32),
            pltpu.VMEM_SHARED((N2,), jnp.float32),
            pltpu.SemaphoreType.DMA,
        ],
    )
    return f(edge_index, planes3, batch)


def kernel(x, edge_index, batch, W_l, W_r, b1, gamma, beta, W_lin, b_lin):
    b1_b = jnp.broadcast_to(b1[None, :], (8, H))
    xr = _dense_xr(x, W_r, b1_b)
    aggp = _sage_sc(x, edge_index)

    degc = (aggp[0, N:, :].reshape(DR * D)
            + aggp[1, N:, :].reshape(DR * D))[:N].reshape(N, 1)

    hpre, stats = _dense_a(aggp, xr, degc, W_l)

    gamma_b = jnp.broadcast_to(gamma[None, :], (8, H))
    beta_b = jnp.broadcast_to(beta[None, :], (8, H))
    wlin_pad = jnp.zeros((8, H), jnp.float32).at[:OUT].set(W_lin)
    planes3 = _dense_b(hpre, stats, degc, gamma_b, beta_b, wlin_pad)

    pool, cnt = _appnp_sc(edge_index, planes3, batch)
    pool = pool.reshape(NC, G)
    cnt = cnt.reshape(G)

    out = (pool / jnp.clip(cnt, 1.0, None)[None, :]).T + b_lin[None, :]
    return out
